# baseline (speedup 1.0000x reference)
import functools

import jax
import jax.numpy as jnp
from jax import lax
from jax.experimental import pallas as pl
from jax.experimental.pallas import tpu as pltpu

N = 2048
D = 1024
NH = 16
HD = 64
E = 8
K = 2
HID = 1024
TM = 256
P = 6144
NT = P // TM
NEG = -1e30


def _k1_body(x_ref, s_ref, b_ref, w_ref, out_ref):
    x = x_ref[...]
    mean = jnp.mean(x, axis=-1, keepdims=True)
    var = jnp.mean((x - mean) ** 2, axis=-1, keepdims=True)
    h = (x - mean) * jax.lax.rsqrt(var + 1e-5) * s_ref[...] + b_ref[...]
    out_ref[...] = jnp.dot(h, w_ref[...], preferred_element_type=jnp.float32)


def _k1(x, ln1_s, ln1_b, qkv_w):
    return pl.pallas_call(
        _k1_body,
        grid=(8, 3),
        in_specs=[
            pl.BlockSpec((256, D), lambda i, j: (i, 0)),
            pl.BlockSpec((1, D), lambda i, j: (0, 0)),
            pl.BlockSpec((1, D), lambda i, j: (0, 0)),
            pl.BlockSpec((D, 1024), lambda i, j: (0, j)),
        ],
        out_specs=pl.BlockSpec((256, 1024), lambda i, j: (i, j)),
        out_shape=jax.ShapeDtypeStruct((N, 3 * D), jnp.float32),
    )(x, ln1_s.reshape(1, D), ln1_b.reshape(1, D), qkv_w)


def _k2_body(q_ref, k_ref, v_ref, o_ref):
    scale = HD ** -0.5
    for h in range(2):
        sl = slice(h * HD, (h + 1) * HD)
        q = q_ref[:, sl]
        k = k_ref[:, sl]
        v = v_ref[:, sl]
        s = lax.dot_general(q, k, (((1,), (1,)), ((), ())),
                            preferred_element_type=jnp.float32) * scale
        m = jnp.max(s, axis=-1, keepdims=True)
        e = jnp.exp(s - m)
        p = e / jnp.sum(e, axis=-1, keepdims=True)
        o_ref[:, sl] = jnp.dot(p, v, preferred_element_type=jnp.float32)


def _k2(qkv):
    return pl.pallas_call(
        _k2_body,
        grid=(8, 8),
        in_specs=[
            pl.BlockSpec((256, 128), lambda p, t: (t, p)),
            pl.BlockSpec((N, 128), lambda p, t: (0, 8 + p)),
            pl.BlockSpec((N, 128), lambda p, t: (0, 16 + p)),
        ],
        out_specs=pl.BlockSpec((256, 128), lambda p, t: (t, p)),
        out_shape=jax.ShapeDtypeStruct((N, D), jnp.float32),
    )(qkv, qkv, qkv)


def _k3_body(o_ref, x_ref, pw_ref, pb_ref, s_ref, b_ref, gw_ref,
             x1_ref, t_ref, e0_ref, e1_ref, p0_ref, p1_ref, imp_ref, load_ref):
    i = pl.program_id(0)
    x1 = x_ref[...] + jnp.dot(o_ref[...], pw_ref[...],
                              preferred_element_type=jnp.float32) + pb_ref[...]
    x1_ref[...] = x1
    mean = jnp.mean(x1, axis=-1, keepdims=True)
    var = jnp.mean((x1 - mean) ** 2, axis=-1, keepdims=True)
    t = (x1 - mean) * jax.lax.rsqrt(var + 1e-5) * s_ref[...] + b_ref[...]
    t_ref[...] = t
    logits = jnp.dot(t, gw_ref[...], preferred_element_type=jnp.float32)
    lane = lax.broadcasted_iota(jnp.int32, logits.shape, 1)
    v = jnp.where(lane < E, logits, NEG)
    m1 = jnp.max(v, axis=-1, keepdims=True)
    e0 = jnp.min(jnp.where(v == m1, lane, 128), axis=-1, keepdims=True)
    v2 = jnp.where(lane == e0, NEG, v)
    m2 = jnp.max(v2, axis=-1, keepdims=True)
    e1 = jnp.min(jnp.where(v2 == m2, lane, 128), axis=-1, keepdims=True)
    bexp = jnp.exp(m2 - m1)
    p0 = 1.0 / (1.0 + bexp)
    p1 = bexp / (1.0 + bexp)
    e0_ref[...] = e0.reshape(1, 256, 1)
    e1_ref[...] = e1.reshape(1, 256, 1)
    p0_ref[...] = p0.reshape(1, 256, 1)
    p1_ref[...] = p1.reshape(1, 256, 1)
    oh0 = jnp.where(lane == e0, 1.0, 0.0)
    oh1 = jnp.where(lane == e1, 1.0, 0.0)
    imp = jnp.sum(oh0 * p0 + oh1 * p1, axis=0, keepdims=True)
    ld = jnp.sum(oh0 * jnp.where(p0 > 0, 1.0, 0.0)
                 + oh1 * jnp.where(p1 > 0, 1.0, 0.0), axis=0, keepdims=True)

    @pl.when(i == 0)
    def _():
        imp_ref[...] = jnp.zeros_like(imp_ref)
        load_ref[...] = jnp.zeros_like(load_ref)

    imp_ref[...] += imp
    load_ref[...] += ld


def _k3(o, x, proj_w, proj_b, ln2_s, ln2_b, gate_w):
    gw = jnp.pad(gate_w, ((0, 0), (0, 128 - E)))
    return pl.pallas_call(
        _k3_body,
        grid=(8,),
        in_specs=[
            pl.BlockSpec((256, D), lambda i: (i, 0)),
            pl.BlockSpec((256, D), lambda i: (i, 0)),
            pl.BlockSpec((D, D), lambda i: (0, 0)),
            pl.BlockSpec((1, D), lambda i: (0, 0)),
            pl.BlockSpec((1, D), lambda i: (0, 0)),
            pl.BlockSpec((1, D), lambda i: (0, 0)),
            pl.BlockSpec((D, 128), lambda i: (0, 0)),
        ],
        out_specs=[
            pl.BlockSpec((256, D), lambda i: (i, 0)),
            pl.BlockSpec((256, D), lambda i: (i, 0)),
            pl.BlockSpec((1, 256, 1), lambda i: (i, 0, 0)),
            pl.BlockSpec((1, 256, 1), lambda i: (i, 0, 0)),
            pl.BlockSpec((1, 256, 1), lambda i: (i, 0, 0)),
            pl.BlockSpec((1, 256, 1), lambda i: (i, 0, 0)),
            pl.BlockSpec((1, 128), lambda i: (0, 0)),
            pl.BlockSpec((1, 128), lambda i: (0, 0)),
        ],
        out_shape=[
            jax.ShapeDtypeStruct((N, D), jnp.float32),
            jax.ShapeDtypeStruct((N, D), jnp.float32),
            jax.ShapeDtypeStruct((8, 256, 1), jnp.int32),
            jax.ShapeDtypeStruct((8, 256, 1), jnp.int32),
            jax.ShapeDtypeStruct((8, 256, 1), jnp.float32),
            jax.ShapeDtypeStruct((8, 256, 1), jnp.float32),
            jax.ShapeDtypeStruct((1, 128), jnp.float32),
            jax.ShapeDtypeStruct((1, 128), jnp.float32),
        ],
    )(o, x, proj_w, proj_b.reshape(1, D), ln2_s.reshape(1, D),
      ln2_b.reshape(1, D), gw)


def _route_stub(e0, e1, t):
    ex = jnp.concatenate([e0, e1])
    tok = jnp.concatenate([jnp.arange(N), jnp.arange(N)])
    counts = jnp.bincount(ex, length=E)
    padded = ((counts + TM - 1) // TM) * TM
    pstart = jnp.cumsum(padded) - padded
    order = jnp.argsort(ex, stable=True)
    rank = jnp.arange(2 * N) - jnp.take(pstart * 0 + (jnp.cumsum(counts) - counts), ex[order])
    dest_sorted = jnp.take(pstart, ex[order]) + rank
    dest = jnp.zeros((2 * N,), jnp.int32).at[order].set(dest_sorted.astype(jnp.int32))
    d0, d1 = dest[:N], dest[N:]
    src = jnp.zeros((P,), jnp.int32).at[dest].set(tok.astype(jnp.int32))
    nv = jnp.sum(padded) // TM
    tid = jnp.arange(NT)
    pend = pstart + padded
    texp = jnp.zeros((NT,), jnp.int32)
    for e in range(E):
        texp = jnp.where((tid * TM >= pstart[e]) & (tid * TM < pend[e]), e, texp)
    last_e = jnp.max(jnp.where(counts > 0, jnp.arange(E), 0))
    texp = jnp.where(tid >= nv, last_e, texp).astype(jnp.int32)
    meta = jnp.zeros((8,), jnp.int32).at[0].set(nv.astype(jnp.int32))
    xg = jnp.take(t, src, axis=0)
    return d0, d1, texp, meta, xg


def _gather_stub(yg, d0, d1):
    return jnp.take(yg, d0, axis=0), jnp.take(yg, d1, axis=0)


def _k5_body(texp_ref, meta_ref, xg_ref, w1_ref, b1_ref, w2_ref, b2_ref, yg_ref):
    i = pl.program_id(0)

    @pl.when(i < meta_ref[0])
    def _():
        h = jnp.dot(xg_ref[...], w1_ref[0],
                    preferred_element_type=jnp.float32) + b1_ref[0]
        h = jax.nn.gelu(h)
        yg_ref[...] = jnp.dot(h, w2_ref[0],
                              preferred_element_type=jnp.float32) + b2_ref[0]


def _k5(texp, meta, xg, w1, b1, w2, b2):
    def xg_idx(i, texp_ref, meta_ref):
        return (jnp.minimum(i, meta_ref[0] - 1), 0)

    def w_idx(i, texp_ref, meta_ref):
        return (texp_ref[jnp.minimum(i, meta_ref[0] - 1)], 0, 0)

    def b_idx(i, texp_ref, meta_ref):
        return (texp_ref[jnp.minimum(i, meta_ref[0] - 1)], 0, 0)

    return pl.pallas_call(
        _k5_body,
        grid_spec=pltpu.PrefetchScalarGridSpec(
            num_scalar_prefetch=2,
            grid=(NT,),
            in_specs=[
                pl.BlockSpec((TM, D), xg_idx),
                pl.BlockSpec((1, D, HID), w_idx),
                pl.BlockSpec((1, 1, HID), b_idx),
                pl.BlockSpec((1, HID, D), w_idx),
                pl.BlockSpec((1, 1, D), b_idx),
            ],
            out_specs=pl.BlockSpec((TM, D), xg_idx),
        ),
        out_shape=jax.ShapeDtypeStruct((P, D), jnp.float32),
    )(texp, meta, xg, w1, b1.reshape(E, 1, HID), w2, b2.reshape(E, 1, D))


def _k7_body(x1_ref, g0_ref, g1_ref, p0_ref, p1_ref, out_ref):
    p0 = p0_ref[...].reshape(256, 1)
    p1 = p1_ref[...].reshape(256, 1)
    out_ref[...] = x1_ref[...] + p0 * g0_ref[...] + p1 * g1_ref[...]


def _k7(x1, g0, g1, p0c, p1c):
    return pl.pallas_call(
        _k7_body,
        grid=(8,),
        in_specs=[
            pl.BlockSpec((256, D), lambda i: (i, 0)),
            pl.BlockSpec((256, D), lambda i: (i, 0)),
            pl.BlockSpec((256, D), lambda i: (i, 0)),
            pl.BlockSpec((1, 256, 1), lambda i: (i, 0, 0)),
            pl.BlockSpec((1, 256, 1), lambda i: (i, 0, 0)),
        ],
        out_specs=pl.BlockSpec((256, D), lambda i: (i, 0)),
        out_shape=jax.ShapeDtypeStruct((N, D), jnp.float32),
    )(x1, g0, g1, p0c, p1c)


def kernel(x, ln1_scale, ln1_bias, qkv_w, proj_w, proj_b,
           ln2_scale, ln2_bias, gate_w, w1, b1, w2, b2):
    x2 = x.reshape(N, D)
    qkv = _k1(x2, ln1_scale, ln1_bias, qkv_w)
    o = _k2(qkv)
    x1, t, e0c, e1c, p0c, p1c, imp, load = _k3(
        o, x2, proj_w, proj_b, ln2_scale, ln2_bias, gate_w)
    e0 = e0c.reshape(N)
    e1 = e1c.reshape(N)
    d0, d1, texp, meta, xg = _route_stub(e0, e1, t)
    yg = _k5(texp, meta, xg, w1, b1, w2, b2)
    g0, g1 = _gather_stub(yg, d0, d1)
    xo = _k7(x1, g0, g1, p0c, p1c)
    return xo.reshape(1, N, D), imp[0, :E], load[0, :E]


# speedup vs baseline: 1.3143x; 1.3143x over previous
import jax
import jax.numpy as jnp
from jax import lax
from jax.experimental import pallas as pl
from jax.experimental.pallas import tpu as pltpu
from jax.experimental.pallas import tpu_sc as plsc

N = 2048
D = 1024
NH = 16
HD = 64
E = 8
TOPK = 2
HID = 1024
TM = 256
P = 6144
NT = P // TM
NEG = -1e30


def _k1_body(x_ref, s_ref, b_ref, w_ref, out_ref):
    x = x_ref[...]
    mean = jnp.mean(x, axis=-1, keepdims=True)
    var = jnp.mean((x - mean) ** 2, axis=-1, keepdims=True)
    h = (x - mean) * jax.lax.rsqrt(var + 1e-5) * s_ref[...] + b_ref[...]
    out_ref[...] = jnp.dot(h, w_ref[...], preferred_element_type=jnp.float32)


def _k1(x, ln1_s, ln1_b, qkv_w):
    return pl.pallas_call(
        _k1_body,
        grid=(8, 3),
        in_specs=[
            pl.BlockSpec((256, D), lambda i, j: (i, 0)),
            pl.BlockSpec((1, D), lambda i, j: (0, 0)),
            pl.BlockSpec((1, D), lambda i, j: (0, 0)),
            pl.BlockSpec((D, 1024), lambda i, j: (0, j)),
        ],
        out_specs=pl.BlockSpec((256, 1024), lambda i, j: (i, j)),
        out_shape=jax.ShapeDtypeStruct((N, 3 * D), jnp.float32),
    )(x, ln1_s.reshape(1, D), ln1_b.reshape(1, D), qkv_w)


def _k2_body(q_ref, k_ref, v_ref, o_ref):
    scale = HD ** -0.5
    for h in range(2):
        sl = slice(h * HD, (h + 1) * HD)
        q = q_ref[:, sl]
        k = k_ref[:, sl]
        v = v_ref[:, sl]
        s = lax.dot_general(q, k, (((1,), (1,)), ((), ())),
                            preferred_element_type=jnp.float32) * scale
        m = jnp.max(s, axis=-1, keepdims=True)
        e = jnp.exp(s - m)
        p = e / jnp.sum(e, axis=-1, keepdims=True)
        o_ref[:, sl] = jnp.dot(p, v, preferred_element_type=jnp.float32)


def _k2(qkv):
    return pl.pallas_call(
        _k2_body,
        grid=(8, 8),
        in_specs=[
            pl.BlockSpec((256, 128), lambda p, t: (t, p)),
            pl.BlockSpec((N, 128), lambda p, t: (0, 8 + p)),
            pl.BlockSpec((N, 128), lambda p, t: (0, 16 + p)),
        ],
        out_specs=pl.BlockSpec((256, 128), lambda p, t: (t, p)),
        out_shape=jax.ShapeDtypeStruct((N, D), jnp.float32),
    )(qkv, qkv, qkv)


def _k3_body(o_ref, x_ref, pw_ref, pb_ref, s_ref, b_ref, gw_ref,
             x1_ref, t_ref, e0_ref, e1_ref, p0_ref, p1_ref, imp_ref, load_ref,
             r0_ref, r1_ref, cnt_ref):
    i = pl.program_id(0)
    x1 = x_ref[...] + jnp.dot(o_ref[...], pw_ref[...],
                              preferred_element_type=jnp.float32) + pb_ref[...]
    x1_ref[...] = x1
    mean = jnp.mean(x1, axis=-1, keepdims=True)
    var = jnp.mean((x1 - mean) ** 2, axis=-1, keepdims=True)
    t = (x1 - mean) * jax.lax.rsqrt(var + 1e-5) * s_ref[...] + b_ref[...]
    t_ref[...] = t
    logits = jnp.dot(t, gw_ref[...], preferred_element_type=jnp.float32)
    lane = lax.broadcasted_iota(jnp.int32, logits.shape, 1)
    v = jnp.where(lane < E, logits, NEG)
    m1 = jnp.max(v, axis=-1, keepdims=True)
    e0 = jnp.min(jnp.where(v == m1, lane, 128), axis=-1, keepdims=True)
    v2 = jnp.where(lane == e0, NEG, v)
    m2 = jnp.max(v2, axis=-1, keepdims=True)
    e1 = jnp.min(jnp.where(v2 == m2, lane, 128), axis=-1, keepdims=True)
    bexp = jnp.exp(m2 - m1)
    p0 = 1.0 / (1.0 + bexp)
    p1 = bexp / (1.0 + bexp)
    e0_ref[...] = e0.reshape(1, 256, 1)
    e1_ref[...] = e1.reshape(1, 256, 1)
    p0_ref[...] = p0.reshape(1, 256, 1)
    p1_ref[...] = p1.reshape(1, 256, 1)
    oh0 = jnp.where(lane == e0, 1.0, 0.0)
    oh1 = jnp.where(lane == e1, 1.0, 0.0)
    imp = jnp.sum(oh0 * p0 + oh1 * p1, axis=0, keepdims=True)
    ld = jnp.sum(oh0 * jnp.where(p0 > 0, 1.0, 0.0)
                 + oh1 * jnp.where(p1 > 0, 1.0, 0.0), axis=0, keepdims=True)

    @pl.when(i == 0)
    def _():
        imp_ref[...] = jnp.zeros_like(imp_ref)
        load_ref[...] = jnp.zeros_like(load_ref)
        cnt_ref[...] = jnp.zeros_like(cnt_ref)

    imp_ref[...] += imp
    load_ref[...] += ld
    carry = cnt_ref[...].astype(jnp.float32)
    oh = jnp.concatenate([oh0, oh1], axis=0)
    r_i = lax.broadcasted_iota(jnp.int32, (512, 512), 0)
    c_i = lax.broadcasted_iota(jnp.int32, (512, 512), 1)
    tri = jnp.where(r_i > c_i, 1.0, 0.0)
    cum_excl = jnp.dot(tri, oh, preferred_element_type=jnp.float32)
    rank = jnp.sum((cum_excl + carry) * oh, axis=-1, keepdims=True)
    r0_ref[...] = rank[:256].astype(jnp.int32).reshape(1, 256, 1)
    r1_ref[...] = rank[256:].astype(jnp.int32).reshape(1, 256, 1)
    cnt_ref[...] += jnp.sum(oh, axis=0, keepdims=True).astype(jnp.int32)


def _k3(o, x, proj_w, proj_b, ln2_s, ln2_b, gate_w):
    gw = jnp.pad(gate_w, ((0, 0), (0, 128 - E)))
    return pl.pallas_call(
        _k3_body,
        grid=(8,),
        in_specs=[
            pl.BlockSpec((256, D), lambda i: (i, 0)),
            pl.BlockSpec((256, D), lambda i: (i, 0)),
            pl.BlockSpec((D, D), lambda i: (0, 0)),
            pl.BlockSpec((1, D), lambda i: (0, 0)),
            pl.BlockSpec((1, D), lambda i: (0, 0)),
            pl.BlockSpec((1, D), lambda i: (0, 0)),
            pl.BlockSpec((D, 128), lambda i: (0, 0)),
        ],
        out_specs=[
            pl.BlockSpec((256, D), lambda i: (i, 0)),
            pl.BlockSpec((256, D), lambda i: (i, 0)),
            pl.BlockSpec((1, 256, 1), lambda i: (i, 0, 0)),
            pl.BlockSpec((1, 256, 1), lambda i: (i, 0, 0)),
            pl.BlockSpec((1, 256, 1), lambda i: (i, 0, 0)),
            pl.BlockSpec((1, 256, 1), lambda i: (i, 0, 0)),
            pl.BlockSpec((1, 128), lambda i: (0, 0)),
            pl.BlockSpec((1, 128), lambda i: (0, 0)),
            pl.BlockSpec((1, 256, 1), lambda i: (i, 0, 0)),
            pl.BlockSpec((1, 256, 1), lambda i: (i, 0, 0)),
            pl.BlockSpec((1, 128), lambda i: (0, 0)),
        ],
        out_shape=[
            jax.ShapeDtypeStruct((N, D), jnp.float32),
            jax.ShapeDtypeStruct((N, D), jnp.float32),
            jax.ShapeDtypeStruct((8, 256, 1), jnp.int32),
            jax.ShapeDtypeStruct((8, 256, 1), jnp.int32),
            jax.ShapeDtypeStruct((8, 256, 1), jnp.float32),
            jax.ShapeDtypeStruct((8, 256, 1), jnp.float32),
            jax.ShapeDtypeStruct((1, 128), jnp.float32),
            jax.ShapeDtypeStruct((1, 128), jnp.float32),
            jax.ShapeDtypeStruct((8, 256, 1), jnp.int32),
            jax.ShapeDtypeStruct((8, 256, 1), jnp.int32),
            jax.ShapeDtypeStruct((1, 128), jnp.int32),
        ],
    )(o, x, proj_w, proj_b.reshape(1, D), ln2_s.reshape(1, D),
      ln2_b.reshape(1, D), gw)


def _k3b_body(cnt_ref, e0_ref, e1_ref, r0_ref, r1_ref,
              d0_ref, d1_ref, texp_ref, meta_ref):
    i = pl.program_id(0)
    counts = cnt_ref[...]
    padf = (((counts + (TM - 1)) >> 8) << 8).astype(jnp.float32)
    r_i = lax.broadcasted_iota(jnp.int32, (128, 128), 0)
    c_i = lax.broadcasted_iota(jnp.int32, (128, 128), 1)
    tril = jnp.where(r_i < c_i, 1.0, 0.0)
    pstart = jnp.dot(padf, tril,
                     preferred_element_type=jnp.float32)
    e0 = e0_ref[...].reshape(256, 1)
    e1 = e1_ref[...].reshape(256, 1)
    lane = lax.broadcasted_iota(jnp.int32, (256, 128), 1)
    s0 = jnp.sum(jnp.where(lane == e0, pstart, 0.0), axis=-1, keepdims=True)
    s1 = jnp.sum(jnp.where(lane == e1, pstart, 0.0), axis=-1, keepdims=True)
    d0_ref[...] = (r0_ref[...].reshape(256, 1)
                   + s0.astype(jnp.int32)).reshape(1, 256, 1)
    d1_ref[...] = (r1_ref[...].reshape(256, 1)
                   + s1.astype(jnp.int32)).reshape(1, 256, 1)

    @pl.when(i == 0)
    def _():
        lanev = lax.broadcasted_iota(jnp.int32, (1, 128), 1)
        nv = jnp.sum(padf) * (1.0 / TM)
        last_e = jnp.max(jnp.where(counts > 0, lanev, 0))
        tile_row = (r_i * TM).astype(jnp.float32)
        pstart_b = jnp.broadcast_to(pstart, (128, 128))
        pend_b = pstart_b + jnp.broadcast_to(padf, (128, 128))
        inm = jnp.where((tile_row >= pstart_b) & (tile_row < pend_b), 1.0, 0.0)
        texp = jnp.sum(inm * c_i.astype(jnp.float32), axis=-1, keepdims=True)
        tcol = lax.broadcasted_iota(jnp.int32, (128, 1), 0).astype(jnp.float32)
        texp = jnp.where(tcol >= nv, last_e.astype(jnp.float32), texp)
        texp_ref[...] = texp.astype(jnp.int32)
        meta_ref[...] = jnp.where(lanev == 0, nv.astype(jnp.int32), 0)


def _k3b(cnt, e0c, e1c, r0c, r1c):
    return pl.pallas_call(
        _k3b_body,
        grid=(8,),
        in_specs=[
            pl.BlockSpec((1, 128), lambda i: (0, 0)),
            pl.BlockSpec((1, 256, 1), lambda i: (i, 0, 0)),
            pl.BlockSpec((1, 256, 1), lambda i: (i, 0, 0)),
            pl.BlockSpec((1, 256, 1), lambda i: (i, 0, 0)),
            pl.BlockSpec((1, 256, 1), lambda i: (i, 0, 0)),
        ],
        out_specs=[
            pl.BlockSpec((1, 256, 1), lambda i: (i, 0, 0)),
            pl.BlockSpec((1, 256, 1), lambda i: (i, 0, 0)),
            pl.BlockSpec((128, 1), lambda i: (0, 0)),
            pl.BlockSpec((1, 128), lambda i: (0, 0)),
        ],
        out_shape=[
            jax.ShapeDtypeStruct((8, 256, 1), jnp.int32),
            jax.ShapeDtypeStruct((8, 256, 1), jnp.int32),
            jax.ShapeDtypeStruct((128, 1), jnp.int32),
            jax.ShapeDtypeStruct((1, 128), jnp.int32),
        ],
    )(cnt, e0c, e1c, r0c, r1c)


def _mesh():
    return plsc.VectorSubcoreMesh(core_axis_name="c", subcore_axis_name="s")


def _k4_body(d0_hbm, d1_hbm, t_hbm, xg_hbm, idxv, rowsv, sem):
    wid = lax.axis_index("s") * 2 + lax.axis_index("c")
    base = wid * 64
    pltpu.sync_copy(t_hbm.at[pl.ds(base, 64)], rowsv)
    pltpu.sync_copy(d0_hbm.at[pl.ds(base, 64)], idxv)
    pltpu.async_copy(rowsv, xg_hbm.at[idxv], sem).wait()
    pltpu.sync_copy(d1_hbm.at[pl.ds(base, 64)], idxv)
    pltpu.async_copy(rowsv, xg_hbm.at[idxv], sem).wait()


def _k4(d0, d1, t):
    return pl.kernel(
        _k4_body,
        out_type=jax.ShapeDtypeStruct((P, D), jnp.float32),
        mesh=_mesh(),
        scratch_types=[
            pltpu.VMEM((64,), jnp.int32),
            pltpu.VMEM((64, D), jnp.float32),
            pltpu.SemaphoreType.DMA,
        ],
    )(d0, d1, t)


def _k5_body(texp_ref, meta_ref, xg_ref, w1_ref, b1_ref, w2_ref, b2_ref, yg_ref):
    i = pl.program_id(0)

    @pl.when(i < meta_ref[0])
    def _():
        h = jnp.dot(xg_ref[...], w1_ref[0],
                    preferred_element_type=jnp.float32) + b1_ref[0]
        h = jax.nn.gelu(h)
        yg_ref[...] = jnp.dot(h, w2_ref[0],
                              preferred_element_type=jnp.float32) + b2_ref[0]


def _k5(texp, meta, xg, w1, b1, w2, b2):
    def xg_idx(i, texp_ref, meta_ref):
        return (jnp.minimum(i, meta_ref[0] - 1), 0)

    def w_idx(i, texp_ref, meta_ref):
        return (texp_ref[jnp.minimum(i, meta_ref[0] - 1)], 0, 0)

    return pl.pallas_call(
        _k5_body,
        grid_spec=pltpu.PrefetchScalarGridSpec(
            num_scalar_prefetch=2,
            grid=(NT,),
            in_specs=[
                pl.BlockSpec((TM, D), xg_idx),
                pl.BlockSpec((1, D, HID), w_idx),
                pl.BlockSpec((1, 1, HID), w_idx),
                pl.BlockSpec((1, HID, D), w_idx),
                pl.BlockSpec((1, 1, D), w_idx),
            ],
            out_specs=pl.BlockSpec((TM, D), xg_idx),
        ),
        out_shape=jax.ShapeDtypeStruct((P, D), jnp.float32),
    )(texp, meta, xg, w1, b1.reshape(E, 1, HID), w2, b2.reshape(E, 1, D))


def _k6_body(d0_hbm, d1_hbm, yg_hbm, g0_hbm, g1_hbm, idxv, rowsv, sem):
    wid = lax.axis_index("s") * 2 + lax.axis_index("c")
    base = wid * 64
    pltpu.sync_copy(d0_hbm.at[pl.ds(base, 64)], idxv)
    pltpu.async_copy(yg_hbm.at[idxv], rowsv, sem).wait()
    pltpu.sync_copy(rowsv, g0_hbm.at[pl.ds(base, 64)])
    pltpu.sync_copy(d1_hbm.at[pl.ds(base, 64)], idxv)
    pltpu.async_copy(yg_hbm.at[idxv], rowsv, sem).wait()
    pltpu.sync_copy(rowsv, g1_hbm.at[pl.ds(base, 64)])


def _k6(d0, d1, yg):
    return pl.kernel(
        _k6_body,
        out_type=[
            jax.ShapeDtypeStruct((N, D), jnp.float32),
            jax.ShapeDtypeStruct((N, D), jnp.float32),
        ],
        mesh=_mesh(),
        scratch_types=[
            pltpu.VMEM((64,), jnp.int32),
            pltpu.VMEM((64, D), jnp.float32),
            pltpu.SemaphoreType.DMA,
        ],
    )(d0, d1, yg)


def _k7_body(x1_ref, g0_ref, g1_ref, p0_ref, p1_ref, out_ref):
    p0 = p0_ref[...].reshape(256, 1)
    p1 = p1_ref[...].reshape(256, 1)
    out_ref[...] = x1_ref[...] + p0 * g0_ref[...] + p1 * g1_ref[...]


def _k7(x1, g0, g1, p0c, p1c):
    return pl.pallas_call(
        _k7_body,
        grid=(8,),
        in_specs=[
            pl.BlockSpec((256, D), lambda i: (i, 0)),
            pl.BlockSpec((256, D), lambda i: (i, 0)),
            pl.BlockSpec((256, D), lambda i: (i, 0)),
            pl.BlockSpec((1, 256, 1), lambda i: (i, 0, 0)),
            pl.BlockSpec((1, 256, 1), lambda i: (i, 0, 0)),
        ],
        out_specs=pl.BlockSpec((256, D), lambda i: (i, 0)),
        out_shape=jax.ShapeDtypeStruct((N, D), jnp.float32),
    )(x1, g0, g1, p0c, p1c)


def kernel(x, ln1_scale, ln1_bias, qkv_w, proj_w, proj_b,
           ln2_scale, ln2_bias, gate_w, w1, b1, w2, b2):
    x2 = x.reshape(N, D)
    qkv = _k1(x2, ln1_scale, ln1_bias, qkv_w)
    o = _k2(qkv)
    x1, t, e0c, e1c, p0c, p1c, imp, load, r0c, r1c, cnt = _k3(
        o, x2, proj_w, proj_b, ln2_scale, ln2_bias, gate_w)
    d0c, d1c, texpc, metac = _k3b(cnt, e0c, e1c, r0c, r1c)
    d0 = d0c.reshape(N)
    d1 = d1c.reshape(N)
    texp = texpc.reshape(128)
    meta = metac.reshape(128)
    xg = _k4(d0, d1, t)
    yg = _k5(texp, meta, xg, w1, b1, w2, b2)
    g0, g1 = _k6(d0, d1, yg)
    xo = _k7(x1, g0, g1, p0c, p1c)
    return xo.reshape(1, N, D), imp[0, :E], load[0, :E]


# speedup vs baseline: 1.5672x; 1.1924x over previous
import jax
import jax.numpy as jnp
from jax import lax
from jax.experimental import pallas as pl
from jax.experimental.pallas import tpu as pltpu
from jax.experimental.pallas import tpu_sc as plsc

N = 2048
D = 1024
NH = 16
HD = 64
E = 8
TOPK = 2
HID = 1024
TM = 256
P = 6144
NT = P // TM
NEG = -1e30


def _k1_body(x_ref, s_ref, b_ref, w_ref, out_ref):
    x = x_ref[...]
    mean = jnp.mean(x, axis=-1, keepdims=True)
    var = jnp.mean((x - mean) ** 2, axis=-1, keepdims=True)
    h = (x - mean) * jax.lax.rsqrt(var + 1e-5) * s_ref[...] + b_ref[...]
    out_ref[...] = jnp.dot(h, w_ref[...], preferred_element_type=jnp.float32)


def _k1(x, ln1_s, ln1_b, qkv_w):
    return pl.pallas_call(
        _k1_body,
        grid=(8, 3),
        in_specs=[
            pl.BlockSpec((256, D), lambda i, j: (i, 0)),
            pl.BlockSpec((1, D), lambda i, j: (0, 0)),
            pl.BlockSpec((1, D), lambda i, j: (0, 0)),
            pl.BlockSpec((D, 1024), lambda i, j: (0, j)),
        ],
        out_specs=pl.BlockSpec((256, 1024), lambda i, j: (i, j)),
        out_shape=jax.ShapeDtypeStruct((N, 3 * D), jnp.float32),
    )(x, ln1_s.reshape(1, D), ln1_b.reshape(1, D), qkv_w)


def _k2_body(q_ref, k_ref, v_ref, o_ref):
    scale = HD ** -0.5
    for h in range(2):
        sl = slice(h * HD, (h + 1) * HD)
        q = q_ref[:, sl]
        k = k_ref[:, sl]
        v = v_ref[:, sl]
        s = lax.dot_general(q, k, (((1,), (1,)), ((), ())),
                            preferred_element_type=jnp.float32) * scale
        e = jnp.exp(s)
        r = 1.0 / jnp.sum(e, axis=-1, keepdims=True)
        o_ref[:, sl] = jnp.dot(e, v, preferred_element_type=jnp.float32) * r


def _k2(qkv):
    return pl.pallas_call(
        _k2_body,
        grid=(8, 8),
        in_specs=[
            pl.BlockSpec((256, 128), lambda p, t: (t, p)),
            pl.BlockSpec((N, 128), lambda p, t: (0, 8 + p)),
            pl.BlockSpec((N, 128), lambda p, t: (0, 16 + p)),
        ],
        out_specs=pl.BlockSpec((256, 128), lambda p, t: (t, p)),
        out_shape=jax.ShapeDtypeStruct((N, D), jnp.float32),
    )(qkv, qkv, qkv)


def _k3_body(o_ref, x_ref, pw_ref, pb_ref, s_ref, b_ref, gw_ref,
             x1_ref, t_ref, e0_ref, e1_ref, p0_ref, p1_ref, imp_ref, load_ref,
             r0_ref, r1_ref, cnt_ref):
    i = pl.program_id(0)
    x1 = x_ref[...] + jnp.dot(o_ref[...], pw_ref[...],
                              preferred_element_type=jnp.float32) + pb_ref[...]
    x1_ref[...] = x1
    mean = jnp.mean(x1, axis=-1, keepdims=True)
    var = jnp.mean((x1 - mean) ** 2, axis=-1, keepdims=True)
    t = (x1 - mean) * jax.lax.rsqrt(var + 1e-5) * s_ref[...] + b_ref[...]
    t_ref[...] = t
    logits = jnp.dot(t, gw_ref[...], preferred_element_type=jnp.float32)
    lane = lax.broadcasted_iota(jnp.int32, logits.shape, 1)
    v = jnp.where(lane < E, logits, NEG)
    m1 = jnp.max(v, axis=-1, keepdims=True)
    e0 = jnp.min(jnp.where(v == m1, lane, 128), axis=-1, keepdims=True)
    v2 = jnp.where(lane == e0, NEG, v)
    m2 = jnp.max(v2, axis=-1, keepdims=True)
    e1 = jnp.min(jnp.where(v2 == m2, lane, 128), axis=-1, keepdims=True)
    bexp = jnp.exp(m2 - m1)
    p0 = 1.0 / (1.0 + bexp)
    p1 = bexp / (1.0 + bexp)
    e0_ref[...] = e0.reshape(1, 256, 1)
    e1_ref[...] = e1.reshape(1, 256, 1)
    p0_ref[...] = p0.reshape(1, 256, 1)
    p1_ref[...] = p1.reshape(1, 256, 1)
    oh0 = jnp.where(lane == e0, 1.0, 0.0)
    oh1 = jnp.where(lane == e1, 1.0, 0.0)
    imp = jnp.sum(oh0 * p0 + oh1 * p1, axis=0, keepdims=True)
    ld = jnp.sum(oh0 * jnp.where(p0 > 0, 1.0, 0.0)
                 + oh1 * jnp.where(p1 > 0, 1.0, 0.0), axis=0, keepdims=True)

    @pl.when(i == 0)
    def _():
        imp_ref[...] = jnp.zeros_like(imp_ref)
        load_ref[...] = jnp.zeros_like(load_ref)
        cnt_ref[...] = jnp.zeros_like(cnt_ref)

    imp_ref[...] += imp
    load_ref[...] += ld
    carry = cnt_ref[...].astype(jnp.float32)
    oh = jnp.concatenate([oh0, oh1], axis=0)
    r_i = lax.broadcasted_iota(jnp.int32, (512, 512), 0)
    c_i = lax.broadcasted_iota(jnp.int32, (512, 512), 1)
    tri = jnp.where(r_i > c_i, 1.0, 0.0)
    cum_excl = jnp.dot(tri, oh, preferred_element_type=jnp.float32)
    rank = jnp.sum((cum_excl + carry) * oh, axis=-1, keepdims=True)
    r0_ref[...] = rank[:256].astype(jnp.int32).reshape(1, 256, 1)
    r1_ref[...] = rank[256:].astype(jnp.int32).reshape(1, 256, 1)
    cnt_ref[...] += jnp.sum(oh, axis=0, keepdims=True).astype(jnp.int32)


def _k3(o, x, proj_w, proj_b, ln2_s, ln2_b, gate_w):
    gw = jnp.pad(gate_w, ((0, 0), (0, 128 - E)))
    return pl.pallas_call(
        _k3_body,
        grid=(8,),
        in_specs=[
            pl.BlockSpec((256, D), lambda i: (i, 0)),
            pl.BlockSpec((256, D), lambda i: (i, 0)),
            pl.BlockSpec((D, D), lambda i: (0, 0)),
            pl.BlockSpec((1, D), lambda i: (0, 0)),
            pl.BlockSpec((1, D), lambda i: (0, 0)),
            pl.BlockSpec((1, D), lambda i: (0, 0)),
            pl.BlockSpec((D, 128), lambda i: (0, 0)),
        ],
        out_specs=[
            pl.BlockSpec((256, D), lambda i: (i, 0)),
            pl.BlockSpec((256, D), lambda i: (i, 0)),
            pl.BlockSpec((1, 256, 1), lambda i: (i, 0, 0)),
            pl.BlockSpec((1, 256, 1), lambda i: (i, 0, 0)),
            pl.BlockSpec((1, 256, 1), lambda i: (i, 0, 0)),
            pl.BlockSpec((1, 256, 1), lambda i: (i, 0, 0)),
            pl.BlockSpec((1, 128), lambda i: (0, 0)),
            pl.BlockSpec((1, 128), lambda i: (0, 0)),
            pl.BlockSpec((1, 256, 1), lambda i: (i, 0, 0)),
            pl.BlockSpec((1, 256, 1), lambda i: (i, 0, 0)),
            pl.BlockSpec((1, 128), lambda i: (0, 0)),
        ],
        out_shape=[
            jax.ShapeDtypeStruct((N, D), jnp.float32),
            jax.ShapeDtypeStruct((N, D), jnp.float32),
            jax.ShapeDtypeStruct((8, 256, 1), jnp.int32),
            jax.ShapeDtypeStruct((8, 256, 1), jnp.int32),
            jax.ShapeDtypeStruct((8, 256, 1), jnp.float32),
            jax.ShapeDtypeStruct((8, 256, 1), jnp.float32),
            jax.ShapeDtypeStruct((1, 128), jnp.float32),
            jax.ShapeDtypeStruct((1, 128), jnp.float32),
            jax.ShapeDtypeStruct((8, 256, 1), jnp.int32),
            jax.ShapeDtypeStruct((8, 256, 1), jnp.int32),
            jax.ShapeDtypeStruct((1, 128), jnp.int32),
        ],
    )(o, x, proj_w, proj_b.reshape(1, D), ln2_s.reshape(1, D),
      ln2_b.reshape(1, D), gw)


def _k3b_body(cnt_ref, e0_ref, e1_ref, r0_ref, r1_ref,
              d0_ref, d1_ref, texp_ref, meta_ref):
    i = pl.program_id(0)
    counts = cnt_ref[...]
    padf = (((counts + (TM - 1)) >> 8) << 8).astype(jnp.float32)
    r_i = lax.broadcasted_iota(jnp.int32, (128, 128), 0)
    c_i = lax.broadcasted_iota(jnp.int32, (128, 128), 1)
    tril = jnp.where(r_i < c_i, 1.0, 0.0)
    pstart = jnp.dot(padf, tril,
                     preferred_element_type=jnp.float32)
    e0 = e0_ref[...].reshape(256, 1)
    e1 = e1_ref[...].reshape(256, 1)
    lane = lax.broadcasted_iota(jnp.int32, (256, 128), 1)
    s0 = jnp.sum(jnp.where(lane == e0, pstart, 0.0), axis=-1, keepdims=True)
    s1 = jnp.sum(jnp.where(lane == e1, pstart, 0.0), axis=-1, keepdims=True)
    d0_ref[...] = (r0_ref[...].reshape(256, 1)
                   + s0.astype(jnp.int32)).reshape(1, 256, 1)
    d1_ref[...] = (r1_ref[...].reshape(256, 1)
                   + s1.astype(jnp.int32)).reshape(1, 256, 1)

    @pl.when(i == 0)
    def _():
        lanev = lax.broadcasted_iota(jnp.int32, (1, 128), 1)
        nv = jnp.sum(padf) * (1.0 / TM)
        last_e = jnp.max(jnp.where(counts > 0, lanev, 0))
        tile_row = (r_i * TM).astype(jnp.float32)
        pstart_b = jnp.broadcast_to(pstart, (128, 128))
        pend_b = pstart_b + jnp.broadcast_to(padf, (128, 128))
        inm = jnp.where((tile_row >= pstart_b) & (tile_row < pend_b), 1.0, 0.0)
        texp = jnp.sum(inm * c_i.astype(jnp.float32), axis=-1, keepdims=True)
        tcol = lax.broadcasted_iota(jnp.int32, (128, 1), 0).astype(jnp.float32)
        texp = jnp.where(tcol >= nv, last_e.astype(jnp.float32), texp)
        texp_ref[...] = texp.astype(jnp.int32)
        meta_ref[...] = jnp.where(lanev == 0, nv.astype(jnp.int32), 0)


def _k3b(cnt, e0c, e1c, r0c, r1c):
    return pl.pallas_call(
        _k3b_body,
        grid=(8,),
        in_specs=[
            pl.BlockSpec((1, 128), lambda i: (0, 0)),
            pl.BlockSpec((1, 256, 1), lambda i: (i, 0, 0)),
            pl.BlockSpec((1, 256, 1), lambda i: (i, 0, 0)),
            pl.BlockSpec((1, 256, 1), lambda i: (i, 0, 0)),
            pl.BlockSpec((1, 256, 1), lambda i: (i, 0, 0)),
        ],
        out_specs=[
            pl.BlockSpec((1, 256, 1), lambda i: (i, 0, 0)),
            pl.BlockSpec((1, 256, 1), lambda i: (i, 0, 0)),
            pl.BlockSpec((128, 1), lambda i: (0, 0)),
            pl.BlockSpec((1, 128), lambda i: (0, 0)),
        ],
        out_shape=[
            jax.ShapeDtypeStruct((8, 256, 1), jnp.int32),
            jax.ShapeDtypeStruct((8, 256, 1), jnp.int32),
            jax.ShapeDtypeStruct((128, 1), jnp.int32),
            jax.ShapeDtypeStruct((1, 128), jnp.int32),
        ],
    )(cnt, e0c, e1c, r0c, r1c)


def _mesh():
    return plsc.VectorSubcoreMesh(core_axis_name="c", subcore_axis_name="s")


def _k4_body(d0_hbm, d1_hbm, t_hbm, xg_hbm, idxv, rowsv, sem):
    wid = lax.axis_index("s") * 2 + lax.axis_index("c")
    base = wid * 64
    pltpu.sync_copy(t_hbm.at[pl.ds(base, 64)], rowsv)
    pltpu.sync_copy(d0_hbm.at[pl.ds(base, 64)], idxv)
    pltpu.async_copy(rowsv, xg_hbm.at[idxv], sem).wait()
    pltpu.sync_copy(d1_hbm.at[pl.ds(base, 64)], idxv)
    pltpu.async_copy(rowsv, xg_hbm.at[idxv], sem).wait()


def _k4(d0, d1, t):
    return pl.kernel(
        _k4_body,
        out_type=jax.ShapeDtypeStruct((P, D), jnp.float32),
        mesh=_mesh(),
        scratch_types=[
            pltpu.VMEM((64,), jnp.int32),
            pltpu.VMEM((64, D), jnp.float32),
            pltpu.SemaphoreType.DMA,
        ],
    )(d0, d1, t)


def _k5_body(texp_ref, meta_ref, xg_ref, w1_ref, b1_ref, w2_ref, b2_ref, yg_ref):
    i = pl.program_id(0)

    @pl.when(i < meta_ref[0])
    def _():
        xb = xg_ref[...].astype(jnp.bfloat16)
        h = jnp.dot(xb, w1_ref[0],
                    preferred_element_type=jnp.float32) + b1_ref[0]
        hb = jax.nn.gelu(h).astype(jnp.bfloat16)
        yg_ref[...] = jnp.dot(hb, w2_ref[0],
                              preferred_element_type=jnp.float32) + b2_ref[0]


def _k5(texp, meta, xg, w1, b1, w2, b2):
    def xg_idx(i, texp_ref, meta_ref):
        return (jnp.minimum(i, meta_ref[0] - 1), 0)

    def w_idx(i, texp_ref, meta_ref):
        return (texp_ref[jnp.minimum(i, meta_ref[0] - 1)], 0, 0)

    return pl.pallas_call(
        _k5_body,
        grid_spec=pltpu.PrefetchScalarGridSpec(
            num_scalar_prefetch=2,
            grid=(NT,),
            in_specs=[
                pl.BlockSpec((TM, D), xg_idx),
                pl.BlockSpec((1, D, HID), w_idx),
                pl.BlockSpec((1, 1, HID), w_idx),
                pl.BlockSpec((1, HID, D), w_idx),
                pl.BlockSpec((1, 1, D), w_idx),
            ],
            out_specs=pl.BlockSpec((TM, D), xg_idx),
        ),
        out_shape=jax.ShapeDtypeStruct((P, D), jnp.float32),
    )(texp, meta, xg, w1.astype(jnp.bfloat16), b1.reshape(E, 1, HID),
      w2.astype(jnp.bfloat16), b2.reshape(E, 1, D))


def _k6_body(d0_hbm, d1_hbm, yg_hbm, g0_hbm, g1_hbm, idxv, rowsv, sem):
    wid = lax.axis_index("s") * 2 + lax.axis_index("c")
    base = wid * 64
    pltpu.sync_copy(d0_hbm.at[pl.ds(base, 64)], idxv)
    pltpu.async_copy(yg_hbm.at[idxv], rowsv, sem).wait()
    pltpu.sync_copy(rowsv, g0_hbm.at[pl.ds(base, 64)])
    pltpu.sync_copy(d1_hbm.at[pl.ds(base, 64)], idxv)
    pltpu.async_copy(yg_hbm.at[idxv], rowsv, sem).wait()
    pltpu.sync_copy(rowsv, g1_hbm.at[pl.ds(base, 64)])


def _k6(d0, d1, yg):
    return pl.kernel(
        _k6_body,
        out_type=[
            jax.ShapeDtypeStruct((N, D), jnp.float32),
            jax.ShapeDtypeStruct((N, D), jnp.float32),
        ],
        mesh=_mesh(),
        scratch_types=[
            pltpu.VMEM((64,), jnp.int32),
            pltpu.VMEM((64, D), jnp.float32),
            pltpu.SemaphoreType.DMA,
        ],
    )(d0, d1, yg)


def _k7_body(x1_ref, g0_ref, g1_ref, p0_ref, p1_ref, out_ref):
    p0 = p0_ref[...].reshape(256, 1)
    p1 = p1_ref[...].reshape(256, 1)
    out_ref[...] = x1_ref[...] + p0 * g0_ref[...] + p1 * g1_ref[...]


def _k7(x1, g0, g1, p0c, p1c):
    return pl.pallas_call(
        _k7_body,
        grid=(8,),
        in_specs=[
            pl.BlockSpec((256, D), lambda i: (i, 0)),
            pl.BlockSpec((256, D), lambda i: (i, 0)),
            pl.BlockSpec((256, D), lambda i: (i, 0)),
            pl.BlockSpec((1, 256, 1), lambda i: (i, 0, 0)),
            pl.BlockSpec((1, 256, 1), lambda i: (i, 0, 0)),
        ],
        out_specs=pl.BlockSpec((256, D), lambda i: (i, 0)),
        out_shape=jax.ShapeDtypeStruct((N, D), jnp.float32),
    )(x1, g0, g1, p0c, p1c)


def kernel(x, ln1_scale, ln1_bias, qkv_w, proj_w, proj_b,
           ln2_scale, ln2_bias, gate_w, w1, b1, w2, b2):
    x2 = x.reshape(N, D)
    qkv = _k1(x2, ln1_scale, ln1_bias, qkv_w)
    o = _k2(qkv)
    x1, t, e0c, e1c, p0c, p1c, imp, load, r0c, r1c, cnt = _k3(
        o, x2, proj_w, proj_b, ln2_scale, ln2_bias, gate_w)
    d0c, d1c, texpc, metac = _k3b(cnt, e0c, e1c, r0c, r1c)
    d0 = d0c.reshape(N)
    d1 = d1c.reshape(N)
    texp = texpc.reshape(128)
    meta = metac.reshape(128)
    xg = _k4(d0, d1, t)
    yg = _k5(texp, meta, xg, w1, b1, w2, b2)
    g0, g1 = _k6(d0, d1, yg)
    xo = _k7(x1, g0, g1, p0c, p1c)
    return xo.reshape(1, N, D), imp[0, :E], load[0, :E]


# speedup vs baseline: 1.6309x; 1.0406x over previous
import jax
import jax.numpy as jnp
from jax import lax
from jax.experimental import pallas as pl
from jax.experimental.pallas import tpu as pltpu
from jax.experimental.pallas import tpu_sc as plsc

N = 2048
D = 1024
NH = 16
HD = 64
E = 8
TOPK = 2
HID = 1024
TM = 256
P = 6144
NT = P // TM
NEG = -1e30


def _k1_body(x_ref, s_ref, b_ref, w_ref, out_ref):
    x = x_ref[...]
    mean = jnp.mean(x, axis=-1, keepdims=True)
    var = jnp.mean((x - mean) ** 2, axis=-1, keepdims=True)
    h = (x - mean) * jax.lax.rsqrt(var + 1e-5) * s_ref[...] + b_ref[...]
    out_ref[...] = jnp.dot(h, w_ref[...], preferred_element_type=jnp.float32)


def _k1(x, ln1_s, ln1_b, qkv_w):
    return pl.pallas_call(
        _k1_body,
        grid=(8, 3),
        in_specs=[
            pl.BlockSpec((256, D), lambda i, j: (i, 0)),
            pl.BlockSpec((1, D), lambda i, j: (0, 0)),
            pl.BlockSpec((1, D), lambda i, j: (0, 0)),
            pl.BlockSpec((D, 1024), lambda i, j: (0, j)),
        ],
        out_specs=pl.BlockSpec((256, 1024), lambda i, j: (i, j)),
        out_shape=jax.ShapeDtypeStruct((N, 3 * D), jnp.float32),
    )(x, ln1_s.reshape(1, D), ln1_b.reshape(1, D), qkv_w)


def _k2_body(q_ref, k_ref, v_ref, o_ref):
    scale = HD ** -0.5
    for h in range(2):
        sl = slice(h * HD, (h + 1) * HD)
        q = q_ref[:, sl]
        k = k_ref[:, sl]
        v = v_ref[:, sl]
        s = lax.dot_general(q, k, (((1,), (1,)), ((), ())),
                            preferred_element_type=jnp.float32) * scale
        e = jnp.exp(s)
        r = 1.0 / jnp.sum(e, axis=-1, keepdims=True)
        o_ref[:, sl] = jnp.dot(e, v, preferred_element_type=jnp.float32) * r


def _k2(qkv):
    return pl.pallas_call(
        _k2_body,
        grid=(8, 4),
        in_specs=[
            pl.BlockSpec((512, 128), lambda p, t: (t, p)),
            pl.BlockSpec((N, 128), lambda p, t: (0, 8 + p)),
            pl.BlockSpec((N, 128), lambda p, t: (0, 16 + p)),
        ],
        out_specs=pl.BlockSpec((512, 128), lambda p, t: (t, p)),
        out_shape=jax.ShapeDtypeStruct((N, D), jnp.float32),
    )(qkv, qkv, qkv)


def _k3_body(o_ref, x_ref, pw_ref, pb_ref, s_ref, b_ref, gw_ref,
             x1_ref, t_ref, e0_ref, e1_ref, p0_ref, p1_ref, imp_ref, load_ref,
             r0_ref, r1_ref, cnt_ref):
    i = pl.program_id(0)
    x1 = x_ref[...] + jnp.dot(o_ref[...], pw_ref[...],
                              preferred_element_type=jnp.float32) + pb_ref[...]
    x1_ref[...] = x1
    mean = jnp.mean(x1, axis=-1, keepdims=True)
    var = jnp.mean((x1 - mean) ** 2, axis=-1, keepdims=True)
    t = (x1 - mean) * jax.lax.rsqrt(var + 1e-5) * s_ref[...] + b_ref[...]
    t_ref[...] = t
    logits = jnp.dot(t, gw_ref[...], preferred_element_type=jnp.float32)
    lane = lax.broadcasted_iota(jnp.int32, logits.shape, 1)
    v = jnp.where(lane < E, logits, NEG)
    m1 = jnp.max(v, axis=-1, keepdims=True)
    e0 = jnp.min(jnp.where(v == m1, lane, 128), axis=-1, keepdims=True)
    v2 = jnp.where(lane == e0, NEG, v)
    m2 = jnp.max(v2, axis=-1, keepdims=True)
    e1 = jnp.min(jnp.where(v2 == m2, lane, 128), axis=-1, keepdims=True)
    bexp = jnp.exp(m2 - m1)
    p0 = 1.0 / (1.0 + bexp)
    p1 = bexp / (1.0 + bexp)
    e0_ref[...] = e0.reshape(1, 256, 1)
    e1_ref[...] = e1.reshape(1, 256, 1)
    p0_ref[...] = p0.reshape(1, 256, 1)
    p1_ref[...] = p1.reshape(1, 256, 1)
    oh0 = jnp.where(lane == e0, 1.0, 0.0)
    oh1 = jnp.where(lane == e1, 1.0, 0.0)
    imp = jnp.sum(oh0 * p0 + oh1 * p1, axis=0, keepdims=True)
    ld = jnp.sum(oh0 * jnp.where(p0 > 0, 1.0, 0.0)
                 + oh1 * jnp.where(p1 > 0, 1.0, 0.0), axis=0, keepdims=True)

    @pl.when(i == 0)
    def _():
        imp_ref[...] = jnp.zeros_like(imp_ref)
        load_ref[...] = jnp.zeros_like(load_ref)
        cnt_ref[...] = jnp.zeros_like(cnt_ref)

    imp_ref[...] += imp
    load_ref[...] += ld
    carry = cnt_ref[...].astype(jnp.float32)
    oh = jnp.concatenate([oh0, oh1], axis=0)
    r_i = lax.broadcasted_iota(jnp.int32, (512, 512), 0)
    c_i = lax.broadcasted_iota(jnp.int32, (512, 512), 1)
    tri = jnp.where(r_i > c_i, 1.0, 0.0)
    cum_excl = jnp.dot(tri, oh, preferred_element_type=jnp.float32)
    rank = jnp.sum((cum_excl + carry) * oh, axis=-1, keepdims=True)
    r0_ref[...] = rank[:256].astype(jnp.int32).reshape(1, 256, 1)
    r1_ref[...] = rank[256:].astype(jnp.int32).reshape(1, 256, 1)
    cnt_ref[...] += jnp.sum(oh, axis=0, keepdims=True).astype(jnp.int32)


def _k3(o, x, proj_w, proj_b, ln2_s, ln2_b, gate_w):
    gw = jnp.pad(gate_w, ((0, 0), (0, 128 - E)))
    return pl.pallas_call(
        _k3_body,
        grid=(8,),
        in_specs=[
            pl.BlockSpec((256, D), lambda i: (i, 0)),
            pl.BlockSpec((256, D), lambda i: (i, 0)),
            pl.BlockSpec((D, D), lambda i: (0, 0)),
            pl.BlockSpec((1, D), lambda i: (0, 0)),
            pl.BlockSpec((1, D), lambda i: (0, 0)),
            pl.BlockSpec((1, D), lambda i: (0, 0)),
            pl.BlockSpec((D, 128), lambda i: (0, 0)),
        ],
        out_specs=[
            pl.BlockSpec((256, D), lambda i: (i, 0)),
            pl.BlockSpec((256, D), lambda i: (i, 0)),
            pl.BlockSpec((1, 256, 1), lambda i: (i, 0, 0)),
            pl.BlockSpec((1, 256, 1), lambda i: (i, 0, 0)),
            pl.BlockSpec((1, 256, 1), lambda i: (i, 0, 0)),
            pl.BlockSpec((1, 256, 1), lambda i: (i, 0, 0)),
            pl.BlockSpec((1, 128), lambda i: (0, 0)),
            pl.BlockSpec((1, 128), lambda i: (0, 0)),
            pl.BlockSpec((1, 256, 1), lambda i: (i, 0, 0)),
            pl.BlockSpec((1, 256, 1), lambda i: (i, 0, 0)),
            pl.BlockSpec((1, 128), lambda i: (0, 0)),
        ],
        out_shape=[
            jax.ShapeDtypeStruct((N, D), jnp.float32),
            jax.ShapeDtypeStruct((N, D), jnp.float32),
            jax.ShapeDtypeStruct((8, 256, 1), jnp.int32),
            jax.ShapeDtypeStruct((8, 256, 1), jnp.int32),
            jax.ShapeDtypeStruct((8, 256, 1), jnp.float32),
            jax.ShapeDtypeStruct((8, 256, 1), jnp.float32),
            jax.ShapeDtypeStruct((1, 128), jnp.float32),
            jax.ShapeDtypeStruct((1, 128), jnp.float32),
            jax.ShapeDtypeStruct((8, 256, 1), jnp.int32),
            jax.ShapeDtypeStruct((8, 256, 1), jnp.int32),
            jax.ShapeDtypeStruct((1, 128), jnp.int32),
        ],
    )(o, x, proj_w, proj_b.reshape(1, D), ln2_s.reshape(1, D),
      ln2_b.reshape(1, D), gw)


def _k3b_body(cnt_ref, e0_ref, e1_ref, r0_ref, r1_ref,
              d0_ref, d1_ref, texp_ref, meta_ref):
    i = pl.program_id(0)
    counts = cnt_ref[...]
    padf = (((counts + (TM - 1)) >> 8) << 8).astype(jnp.float32)
    r_i = lax.broadcasted_iota(jnp.int32, (128, 128), 0)
    c_i = lax.broadcasted_iota(jnp.int32, (128, 128), 1)
    tril = jnp.where(r_i < c_i, 1.0, 0.0)
    pstart = jnp.dot(padf, tril,
                     preferred_element_type=jnp.float32)
    e0 = e0_ref[...].reshape(256, 1)
    e1 = e1_ref[...].reshape(256, 1)
    lane = lax.broadcasted_iota(jnp.int32, (256, 128), 1)
    s0 = jnp.sum(jnp.where(lane == e0, pstart, 0.0), axis=-1, keepdims=True)
    s1 = jnp.sum(jnp.where(lane == e1, pstart, 0.0), axis=-1, keepdims=True)
    d0_ref[...] = (r0_ref[...].reshape(256, 1)
                   + s0.astype(jnp.int32)).reshape(1, 256, 1)
    d1_ref[...] = (r1_ref[...].reshape(256, 1)
                   + s1.astype(jnp.int32)).reshape(1, 256, 1)

    @pl.when(i == 0)
    def _():
        lanev = lax.broadcasted_iota(jnp.int32, (1, 128), 1)
        nv = jnp.sum(padf) * (1.0 / TM)
        last_e = jnp.max(jnp.where(counts > 0, lanev, 0))
        tile_row = (r_i * TM).astype(jnp.float32)
        pstart_b = jnp.broadcast_to(pstart, (128, 128))
        pend_b = pstart_b + jnp.broadcast_to(padf, (128, 128))
        inm = jnp.where((tile_row >= pstart_b) & (tile_row < pend_b), 1.0, 0.0)
        texp = jnp.sum(inm * c_i.astype(jnp.float32), axis=-1, keepdims=True)
        tcol = lax.broadcasted_iota(jnp.int32, (128, 1), 0).astype(jnp.float32)
        texp = jnp.where(tcol >= nv, last_e.astype(jnp.float32), texp)
        texp_ref[...] = texp.astype(jnp.int32)
        meta_ref[...] = jnp.where(lanev == 0, nv.astype(jnp.int32), 0)


def _k3b(cnt, e0c, e1c, r0c, r1c):
    return pl.pallas_call(
        _k3b_body,
        grid=(8,),
        in_specs=[
            pl.BlockSpec((1, 128), lambda i: (0, 0)),
            pl.BlockSpec((1, 256, 1), lambda i: (i, 0, 0)),
            pl.BlockSpec((1, 256, 1), lambda i: (i, 0, 0)),
            pl.BlockSpec((1, 256, 1), lambda i: (i, 0, 0)),
            pl.BlockSpec((1, 256, 1), lambda i: (i, 0, 0)),
        ],
        out_specs=[
            pl.BlockSpec((1, 256, 1), lambda i: (i, 0, 0)),
            pl.BlockSpec((1, 256, 1), lambda i: (i, 0, 0)),
            pl.BlockSpec((128, 1), lambda i: (0, 0)),
            pl.BlockSpec((1, 128), lambda i: (0, 0)),
        ],
        out_shape=[
            jax.ShapeDtypeStruct((8, 256, 1), jnp.int32),
            jax.ShapeDtypeStruct((8, 256, 1), jnp.int32),
            jax.ShapeDtypeStruct((128, 1), jnp.int32),
            jax.ShapeDtypeStruct((1, 128), jnp.int32),
        ],
    )(cnt, e0c, e1c, r0c, r1c)


def _mesh():
    return plsc.VectorSubcoreMesh(core_axis_name="c", subcore_axis_name="s")


def _k4_body(d0_hbm, d1_hbm, t_hbm, xg_hbm, idxv, rowsv, sem):
    wid = lax.axis_index("s") * 2 + lax.axis_index("c")
    base = wid * 64
    pltpu.sync_copy(t_hbm.at[pl.ds(base, 64)], rowsv)
    pltpu.sync_copy(d0_hbm.at[pl.ds(base, 64)], idxv)
    pltpu.async_copy(rowsv, xg_hbm.at[idxv], sem).wait()
    pltpu.sync_copy(d1_hbm.at[pl.ds(base, 64)], idxv)
    pltpu.async_copy(rowsv, xg_hbm.at[idxv], sem).wait()


def _k4(d0, d1, t):
    return pl.kernel(
        _k4_body,
        out_type=jax.ShapeDtypeStruct((P, D), jnp.float32),
        mesh=_mesh(),
        scratch_types=[
            pltpu.VMEM((64,), jnp.int32),
            pltpu.VMEM((64, D), jnp.float32),
            pltpu.SemaphoreType.DMA,
        ],
    )(d0, d1, t)


def _k5_body(texp_ref, meta_ref, xg_ref, w1_ref, b1_ref, w2_ref, b2_ref, yg_ref):
    i = pl.program_id(0)

    @pl.when(i < meta_ref[0])
    def _():
        xb = xg_ref[...].astype(jnp.bfloat16)
        h = jnp.dot(xb, w1_ref[0],
                    preferred_element_type=jnp.float32) + b1_ref[0]
        hb = jax.nn.gelu(h).astype(jnp.bfloat16)
        yg_ref[...] = jnp.dot(hb, w2_ref[0],
                              preferred_element_type=jnp.float32) + b2_ref[0]


def _k5(texp, meta, xg, w1, b1, w2, b2):
    def xg_idx(i, texp_ref, meta_ref):
        return (jnp.minimum(i, meta_ref[0] - 1), 0)

    def w_idx(i, texp_ref, meta_ref):
        return (texp_ref[jnp.minimum(i, meta_ref[0] - 1)], 0, 0)

    return pl.pallas_call(
        _k5_body,
        grid_spec=pltpu.PrefetchScalarGridSpec(
            num_scalar_prefetch=2,
            grid=(NT,),
            in_specs=[
                pl.BlockSpec((TM, D), xg_idx),
                pl.BlockSpec((1, D, HID), w_idx),
                pl.BlockSpec((1, 1, HID), w_idx),
                pl.BlockSpec((1, HID, D), w_idx),
                pl.BlockSpec((1, 1, D), w_idx),
            ],
            out_specs=pl.BlockSpec((TM, D), xg_idx),
        ),
        out_shape=jax.ShapeDtypeStruct((P, D), jnp.float32),
    )(texp, meta, xg, w1.astype(jnp.bfloat16), b1.reshape(E, 1, HID),
      w2.astype(jnp.bfloat16), b2.reshape(E, 1, D))


def _k6_body(d0_hbm, d1_hbm, yg_hbm, g0_hbm, g1_hbm, idxv, rowsv, sem):
    wid = lax.axis_index("s") * 2 + lax.axis_index("c")
    base = wid * 64
    pltpu.sync_copy(d0_hbm.at[pl.ds(base, 64)], idxv)
    pltpu.async_copy(yg_hbm.at[idxv], rowsv, sem).wait()
    pltpu.sync_copy(rowsv, g0_hbm.at[pl.ds(base, 64)])
    pltpu.sync_copy(d1_hbm.at[pl.ds(base, 64)], idxv)
    pltpu.async_copy(yg_hbm.at[idxv], rowsv, sem).wait()
    pltpu.sync_copy(rowsv, g1_hbm.at[pl.ds(base, 64)])


def _k6(d0, d1, yg):
    return pl.kernel(
        _k6_body,
        out_type=[
            jax.ShapeDtypeStruct((N, D), jnp.float32),
            jax.ShapeDtypeStruct((N, D), jnp.float32),
        ],
        mesh=_mesh(),
        scratch_types=[
            pltpu.VMEM((64,), jnp.int32),
            pltpu.VMEM((64, D), jnp.float32),
            pltpu.SemaphoreType.DMA,
        ],
    )(d0, d1, yg)


def _k7_body(x1_ref, g0_ref, g1_ref, p0_ref, p1_ref, out_ref):
    p0 = p0_ref[...].reshape(256, 1)
    p1 = p1_ref[...].reshape(256, 1)
    out_ref[...] = x1_ref[...] + p0 * g0_ref[...] + p1 * g1_ref[...]


def _k7(x1, g0, g1, p0c, p1c):
    return pl.pallas_call(
        _k7_body,
        grid=(8,),
        in_specs=[
            pl.BlockSpec((256, D), lambda i: (i, 0)),
            pl.BlockSpec((256, D), lambda i: (i, 0)),
            pl.BlockSpec((256, D), lambda i: (i, 0)),
            pl.BlockSpec((1, 256, 1), lambda i: (i, 0, 0)),
            pl.BlockSpec((1, 256, 1), lambda i: (i, 0, 0)),
        ],
        out_specs=pl.BlockSpec((256, D), lambda i: (i, 0)),
        out_shape=jax.ShapeDtypeStruct((N, D), jnp.float32),
    )(x1, g0, g1, p0c, p1c)


def kernel(x, ln1_scale, ln1_bias, qkv_w, proj_w, proj_b,
           ln2_scale, ln2_bias, gate_w, w1, b1, w2, b2):
    x2 = x.reshape(N, D)
    qkv = _k1(x2, ln1_scale, ln1_bias, qkv_w)
    o = _k2(qkv)
    x1, t, e0c, e1c, p0c, p1c, imp, load, r0c, r1c, cnt = _k3(
        o, x2, proj_w, proj_b, ln2_scale, ln2_bias, gate_w)
    d0c, d1c, texpc, metac = _k3b(cnt, e0c, e1c, r0c, r1c)
    d0 = d0c.reshape(N)
    d1 = d1c.reshape(N)
    texp = texpc.reshape(128)
    meta = metac.reshape(128)
    xg = _k4(d0, d1, t)
    yg = _k5(texp, meta, xg, w1, b1, w2, b2)
    g0, g1 = _k6(d0, d1, yg)
    xo = _k7(x1, g0, g1, p0c, p1c)
    return xo.reshape(1, N, D), imp[0, :E], load[0, :E]


# speedup vs baseline: 1.7592x; 1.0787x over previous
import jax
import jax.numpy as jnp
from jax import lax
from jax.experimental import pallas as pl
from jax.experimental.pallas import tpu as pltpu
from jax.experimental.pallas import tpu_sc as plsc

N = 2048
D = 1024
NH = 16
HD = 64
E = 8
TOPK = 2
HID = 1024
TM = 256
P = 6144
NT = P // TM
NEG = -1e30


def _k1_body(x_ref, s_ref, b_ref, w_ref, out_ref):
    x = x_ref[...]
    mean = jnp.mean(x, axis=-1, keepdims=True)
    var = jnp.mean((x - mean) ** 2, axis=-1, keepdims=True)
    h = (x - mean) * jax.lax.rsqrt(var + 1e-5) * s_ref[...] + b_ref[...]
    out_ref[...] = jnp.dot(h, w_ref[...], preferred_element_type=jnp.float32)


def _k1(x, ln1_s, ln1_b, qkv_w):
    return pl.pallas_call(
        _k1_body,
        grid=(8, 3),
        in_specs=[
            pl.BlockSpec((256, D), lambda i, j: (i, 0)),
            pl.BlockSpec((1, D), lambda i, j: (0, 0)),
            pl.BlockSpec((1, D), lambda i, j: (0, 0)),
            pl.BlockSpec((D, 1024), lambda i, j: (0, j)),
        ],
        out_specs=pl.BlockSpec((256, 1024), lambda i, j: (i, j)),
        out_shape=jax.ShapeDtypeStruct((N, 3 * D), jnp.float32),
    )(x, ln1_s.reshape(1, D), ln1_b.reshape(1, D), qkv_w)


def _k2_body(q_ref, k_ref, v_ref, o_ref):
    scale = HD ** -0.5
    for h in range(2):
        sl = slice(h * HD, (h + 1) * HD)
        q = q_ref[:, sl]
        k = k_ref[:, sl]
        v = v_ref[:, sl]
        s = lax.dot_general(q, k, (((1,), (1,)), ((), ())),
                            preferred_element_type=jnp.float32) * scale
        e = jnp.exp(s)
        r = 1.0 / jnp.sum(e, axis=-1, keepdims=True)
        o_ref[:, sl] = jnp.dot(e, v, preferred_element_type=jnp.float32) * r


def _k2(qkv):
    return pl.pallas_call(
        _k2_body,
        grid=(8, 4),
        in_specs=[
            pl.BlockSpec((512, 128), lambda p, t: (t, p)),
            pl.BlockSpec((N, 128), lambda p, t: (0, 8 + p)),
            pl.BlockSpec((N, 128), lambda p, t: (0, 16 + p)),
        ],
        out_specs=pl.BlockSpec((512, 128), lambda p, t: (t, p)),
        out_shape=jax.ShapeDtypeStruct((N, D), jnp.float32),
    )(qkv, qkv, qkv)


def _k3_body(o_ref, x_ref, pw_ref, pb_ref, s_ref, b_ref, gw_ref,
             x1_ref, t_ref, e0_ref, e1_ref, p0_ref, p1_ref, imp_ref, load_ref,
             r0_ref, r1_ref, cnt_ref):
    i = pl.program_id(0)
    x1 = x_ref[...] + jnp.dot(o_ref[...], pw_ref[...],
                              preferred_element_type=jnp.float32) + pb_ref[...]
    x1_ref[...] = x1
    mean = jnp.mean(x1, axis=-1, keepdims=True)
    var = jnp.mean((x1 - mean) ** 2, axis=-1, keepdims=True)
    t = (x1 - mean) * jax.lax.rsqrt(var + 1e-5) * s_ref[...] + b_ref[...]
    t_ref[...] = t
    logits = jnp.dot(t, gw_ref[...], preferred_element_type=jnp.float32)
    lane = lax.broadcasted_iota(jnp.int32, logits.shape, 1)
    v = jnp.where(lane < E, logits, NEG)
    m1 = jnp.max(v, axis=-1, keepdims=True)
    e0 = jnp.min(jnp.where(v == m1, lane, 128), axis=-1, keepdims=True)
    v2 = jnp.where(lane == e0, NEG, v)
    m2 = jnp.max(v2, axis=-1, keepdims=True)
    e1 = jnp.min(jnp.where(v2 == m2, lane, 128), axis=-1, keepdims=True)
    bexp = jnp.exp(m2 - m1)
    p0 = 1.0 / (1.0 + bexp)
    p1 = bexp / (1.0 + bexp)
    e0_ref[...] = e0.reshape(1, 256, 1)
    e1_ref[...] = e1.reshape(1, 256, 1)
    p0_ref[...] = p0.reshape(1, 256, 1)
    p1_ref[...] = p1.reshape(1, 256, 1)
    oh0 = jnp.where(lane == e0, 1.0, 0.0)
    oh1 = jnp.where(lane == e1, 1.0, 0.0)
    imp = jnp.sum(oh0 * p0 + oh1 * p1, axis=0, keepdims=True)
    ld = jnp.sum(oh0 * jnp.where(p0 > 0, 1.0, 0.0)
                 + oh1 * jnp.where(p1 > 0, 1.0, 0.0), axis=0, keepdims=True)

    @pl.when(i == 0)
    def _():
        imp_ref[...] = jnp.zeros_like(imp_ref)
        load_ref[...] = jnp.zeros_like(load_ref)
        cnt_ref[...] = jnp.zeros_like(cnt_ref)

    imp_ref[...] += imp
    load_ref[...] += ld
    carry = cnt_ref[...].astype(jnp.float32)
    oh = jnp.concatenate([oh0, oh1], axis=0)
    r_i = lax.broadcasted_iota(jnp.int32, (512, 512), 0)
    c_i = lax.broadcasted_iota(jnp.int32, (512, 512), 1)
    tri = jnp.where(r_i > c_i, 1.0, 0.0)
    cum_excl = jnp.dot(tri, oh, preferred_element_type=jnp.float32)
    rank = jnp.sum((cum_excl + carry) * oh, axis=-1, keepdims=True)
    r0_ref[...] = rank[:256].astype(jnp.int32).reshape(1, 256, 1)
    r1_ref[...] = rank[256:].astype(jnp.int32).reshape(1, 256, 1)
    cnt_ref[...] += jnp.sum(oh, axis=0, keepdims=True).astype(jnp.int32)


def _k3(o, x, proj_w, proj_b, ln2_s, ln2_b, gate_w):
    gw = jnp.pad(gate_w, ((0, 0), (0, 128 - E)))
    return pl.pallas_call(
        _k3_body,
        grid=(8,),
        in_specs=[
            pl.BlockSpec((256, D), lambda i: (i, 0)),
            pl.BlockSpec((256, D), lambda i: (i, 0)),
            pl.BlockSpec((D, D), lambda i: (0, 0)),
            pl.BlockSpec((1, D), lambda i: (0, 0)),
            pl.BlockSpec((1, D), lambda i: (0, 0)),
            pl.BlockSpec((1, D), lambda i: (0, 0)),
            pl.BlockSpec((D, 128), lambda i: (0, 0)),
        ],
        out_specs=[
            pl.BlockSpec((256, D), lambda i: (i, 0)),
            pl.BlockSpec((256, D), lambda i: (i, 0)),
            pl.BlockSpec((1, 256, 1), lambda i: (i, 0, 0)),
            pl.BlockSpec((1, 256, 1), lambda i: (i, 0, 0)),
            pl.BlockSpec((1, 256, 1), lambda i: (i, 0, 0)),
            pl.BlockSpec((1, 256, 1), lambda i: (i, 0, 0)),
            pl.BlockSpec((1, 128), lambda i: (0, 0)),
            pl.BlockSpec((1, 128), lambda i: (0, 0)),
            pl.BlockSpec((1, 256, 1), lambda i: (i, 0, 0)),
            pl.BlockSpec((1, 256, 1), lambda i: (i, 0, 0)),
            pl.BlockSpec((1, 128), lambda i: (0, 0)),
        ],
        out_shape=[
            jax.ShapeDtypeStruct((N, D), jnp.float32),
            jax.ShapeDtypeStruct((N, D), jnp.float32),
            jax.ShapeDtypeStruct((8, 256, 1), jnp.int32),
            jax.ShapeDtypeStruct((8, 256, 1), jnp.int32),
            jax.ShapeDtypeStruct((8, 256, 1), jnp.float32),
            jax.ShapeDtypeStruct((8, 256, 1), jnp.float32),
            jax.ShapeDtypeStruct((1, 128), jnp.float32),
            jax.ShapeDtypeStruct((1, 128), jnp.float32),
            jax.ShapeDtypeStruct((8, 256, 1), jnp.int32),
            jax.ShapeDtypeStruct((8, 256, 1), jnp.int32),
            jax.ShapeDtypeStruct((1, 128), jnp.int32),
        ],
    )(o, x, proj_w, proj_b.reshape(1, D), ln2_s.reshape(1, D),
      ln2_b.reshape(1, D), gw)


def _k3b_body(cnt_ref, e0_ref, e1_ref, r0_ref, r1_ref,
              d0_ref, d1_ref, texp_ref, meta_ref):
    i = pl.program_id(0)
    counts = cnt_ref[...]
    padf = (((counts + (TM - 1)) >> 8) << 8).astype(jnp.float32)
    r_i = lax.broadcasted_iota(jnp.int32, (128, 128), 0)
    c_i = lax.broadcasted_iota(jnp.int32, (128, 128), 1)
    tril = jnp.where(r_i < c_i, 1.0, 0.0)
    pstart = jnp.dot(padf, tril,
                     preferred_element_type=jnp.float32)
    e0 = e0_ref[...].reshape(256, 1)
    e1 = e1_ref[...].reshape(256, 1)
    lane = lax.broadcasted_iota(jnp.int32, (256, 128), 1)
    s0 = jnp.sum(jnp.where(lane == e0, pstart, 0.0), axis=-1, keepdims=True)
    s1 = jnp.sum(jnp.where(lane == e1, pstart, 0.0), axis=-1, keepdims=True)
    d0_ref[...] = (r0_ref[...].reshape(256, 1)
                   + s0.astype(jnp.int32)).reshape(1, 256, 1)
    d1_ref[...] = (r1_ref[...].reshape(256, 1)
                   + s1.astype(jnp.int32)).reshape(1, 256, 1)

    @pl.when(i == 0)
    def _():
        lanev = lax.broadcasted_iota(jnp.int32, (1, 128), 1)
        nv = jnp.sum(padf) * (1.0 / TM)
        last_e = jnp.max(jnp.where(counts > 0, lanev, 0))
        tile_row = (r_i * TM).astype(jnp.float32)
        pstart_b = jnp.broadcast_to(pstart, (128, 128))
        pend_b = pstart_b + jnp.broadcast_to(padf, (128, 128))
        inm = jnp.where((tile_row >= pstart_b) & (tile_row < pend_b), 1.0, 0.0)
        texp = jnp.sum(inm * c_i.astype(jnp.float32), axis=-1, keepdims=True)
        tcol = lax.broadcasted_iota(jnp.int32, (128, 1), 0).astype(jnp.float32)
        texp = jnp.where(tcol >= nv, last_e.astype(jnp.float32), texp)
        texp_ref[...] = texp.astype(jnp.int32)
        meta_ref[...] = jnp.where(lanev == 0, nv.astype(jnp.int32), 0)


def _k3b(cnt, e0c, e1c, r0c, r1c):
    return pl.pallas_call(
        _k3b_body,
        grid=(8,),
        in_specs=[
            pl.BlockSpec((1, 128), lambda i: (0, 0)),
            pl.BlockSpec((1, 256, 1), lambda i: (i, 0, 0)),
            pl.BlockSpec((1, 256, 1), lambda i: (i, 0, 0)),
            pl.BlockSpec((1, 256, 1), lambda i: (i, 0, 0)),
            pl.BlockSpec((1, 256, 1), lambda i: (i, 0, 0)),
        ],
        out_specs=[
            pl.BlockSpec((1, 256, 1), lambda i: (i, 0, 0)),
            pl.BlockSpec((1, 256, 1), lambda i: (i, 0, 0)),
            pl.BlockSpec((128, 1), lambda i: (0, 0)),
            pl.BlockSpec((1, 128), lambda i: (0, 0)),
        ],
        out_shape=[
            jax.ShapeDtypeStruct((8, 256, 1), jnp.int32),
            jax.ShapeDtypeStruct((8, 256, 1), jnp.int32),
            jax.ShapeDtypeStruct((128, 1), jnp.int32),
            jax.ShapeDtypeStruct((1, 128), jnp.int32),
        ],
    )(cnt, e0c, e1c, r0c, r1c)


def _mesh():
    return plsc.VectorSubcoreMesh(core_axis_name="c", subcore_axis_name="s")


def _k4_body(d0_hbm, d1_hbm, t_hbm, xg_hbm, idxv, rowsv, sem):
    wid = lax.axis_index("s") * 2 + lax.axis_index("c")
    base = wid * 64
    pltpu.sync_copy(t_hbm.at[pl.ds(base, 64)], rowsv)
    pltpu.sync_copy(d0_hbm.at[pl.ds(base, 64)], idxv)
    pltpu.async_copy(rowsv, xg_hbm.at[idxv], sem).wait()
    pltpu.sync_copy(d1_hbm.at[pl.ds(base, 64)], idxv)
    pltpu.async_copy(rowsv, xg_hbm.at[idxv], sem).wait()


def _k4(d0, d1, t):
    return pl.kernel(
        _k4_body,
        out_type=jax.ShapeDtypeStruct((P, D), jnp.float32),
        mesh=_mesh(),
        scratch_types=[
            pltpu.VMEM((64,), jnp.int32),
            pltpu.VMEM((64, D), jnp.float32),
            pltpu.SemaphoreType.DMA,
        ],
    )(d0, d1, t)


def _k5_body(texp_ref, meta_ref, xg_ref, w1_ref, b1_ref, w2_ref, b2_ref, yg_ref):
    i = pl.program_id(0)

    @pl.when(i < meta_ref[0])
    def _():
        xb = xg_ref[...].astype(jnp.bfloat16)
        h = jnp.dot(xb, w1_ref[0].astype(jnp.bfloat16),
                    preferred_element_type=jnp.float32) + b1_ref[0]
        hb = jax.nn.gelu(h).astype(jnp.bfloat16)
        yg_ref[...] = jnp.dot(hb, w2_ref[0].astype(jnp.bfloat16),
                              preferred_element_type=jnp.float32) + b2_ref[0]


def _k5(texp, meta, xg, w1, b1, w2, b2):
    def xg_idx(i, texp_ref, meta_ref):
        return (jnp.minimum(i, meta_ref[0] - 1), 0)

    def w_idx(i, texp_ref, meta_ref):
        return (texp_ref[jnp.minimum(i, meta_ref[0] - 1)], 0, 0)

    return pl.pallas_call(
        _k5_body,
        grid_spec=pltpu.PrefetchScalarGridSpec(
            num_scalar_prefetch=2,
            grid=(NT,),
            in_specs=[
                pl.BlockSpec((TM, D), xg_idx),
                pl.BlockSpec((1, D, HID), w_idx),
                pl.BlockSpec((1, 1, HID), w_idx),
                pl.BlockSpec((1, HID, D), w_idx),
                pl.BlockSpec((1, 1, D), w_idx),
            ],
            out_specs=pl.BlockSpec((TM, D), xg_idx),
        ),
        out_shape=jax.ShapeDtypeStruct((P, D), jnp.float32),
    )(texp, meta, xg, w1, b1.reshape(E, 1, HID), w2, b2.reshape(E, 1, D))


def _k6_body(d0_hbm, d1_hbm, yg_hbm, g0_hbm, g1_hbm, idxv, rowsv, sem):
    wid = lax.axis_index("s") * 2 + lax.axis_index("c")
    base = wid * 64
    pltpu.sync_copy(d0_hbm.at[pl.ds(base, 64)], idxv)
    pltpu.async_copy(yg_hbm.at[idxv], rowsv, sem).wait()
    pltpu.sync_copy(rowsv, g0_hbm.at[pl.ds(base, 64)])
    pltpu.sync_copy(d1_hbm.at[pl.ds(base, 64)], idxv)
    pltpu.async_copy(yg_hbm.at[idxv], rowsv, sem).wait()
    pltpu.sync_copy(rowsv, g1_hbm.at[pl.ds(base, 64)])


def _k6(d0, d1, yg):
    return pl.kernel(
        _k6_body,
        out_type=[
            jax.ShapeDtypeStruct((N, D), jnp.float32),
            jax.ShapeDtypeStruct((N, D), jnp.float32),
        ],
        mesh=_mesh(),
        scratch_types=[
            pltpu.VMEM((64,), jnp.int32),
            pltpu.VMEM((64, D), jnp.float32),
            pltpu.SemaphoreType.DMA,
        ],
    )(d0, d1, yg)


def _k7_body(x1_ref, g0_ref, g1_ref, p0_ref, p1_ref, out_ref):
    p0 = p0_ref[...].reshape(256, 1)
    p1 = p1_ref[...].reshape(256, 1)
    out_ref[...] = x1_ref[...] + p0 * g0_ref[...] + p1 * g1_ref[...]


def _k7(x1, g0, g1, p0c, p1c):
    return pl.pallas_call(
        _k7_body,
        grid=(8,),
        in_specs=[
            pl.BlockSpec((256, D), lambda i: (i, 0)),
            pl.BlockSpec((256, D), lambda i: (i, 0)),
            pl.BlockSpec((256, D), lambda i: (i, 0)),
            pl.BlockSpec((1, 256, 1), lambda i: (i, 0, 0)),
            pl.BlockSpec((1, 256, 1), lambda i: (i, 0, 0)),
        ],
        out_specs=pl.BlockSpec((256, D), lambda i: (i, 0)),
        out_shape=jax.ShapeDtypeStruct((N, D), jnp.float32),
    )(x1, g0, g1, p0c, p1c)


def kernel(x, ln1_scale, ln1_bias, qkv_w, proj_w, proj_b,
           ln2_scale, ln2_bias, gate_w, w1, b1, w2, b2):
    x2 = x.reshape(N, D)
    qkv = _k1(x2, ln1_scale, ln1_bias, qkv_w)
    o = _k2(qkv)
    x1, t, e0c, e1c, p0c, p1c, imp, load, r0c, r1c, cnt = _k3(
        o, x2, proj_w, proj_b, ln2_scale, ln2_bias, gate_w)
    d0c, d1c, texpc, metac = _k3b(cnt, e0c, e1c, r0c, r1c)
    d0 = d0c.reshape(N)
    d1 = d1c.reshape(N)
    texp = texpc.reshape(128)
    meta = metac.reshape(128)
    xg = _k4(d0, d1, t)
    yg = _k5(texp, meta, xg, w1, b1, w2, b2)
    g0, g1 = _k6(d0, d1, yg)
    xo = _k7(x1, g0, g1, p0c, p1c)
    return xo.reshape(1, N, D), imp[0, :E], load[0, :E]


# speedup vs baseline: 1.8695x; 1.0627x over previous
import jax
import jax.numpy as jnp
from jax import lax
from jax.experimental import pallas as pl
from jax.experimental.pallas import tpu as pltpu
from jax.experimental.pallas import tpu_sc as plsc

N = 2048
D = 1024
NH = 16
HD = 64
E = 8
TOPK = 2
HID = 1024
TM = 256
P = 6144
NT = P // TM
NEG = -1e30


def _k1_body(x_ref, s_ref, b_ref, w_ref, out_ref):
    x = x_ref[...]
    mean = jnp.mean(x, axis=-1, keepdims=True)
    var = jnp.mean((x - mean) ** 2, axis=-1, keepdims=True)
    h = (x - mean) * jax.lax.rsqrt(var + 1e-5) * s_ref[...] + b_ref[...]
    out_ref[...] = jnp.dot(h, w_ref[...], preferred_element_type=jnp.float32)


def _k1(x, ln1_s, ln1_b, qkv_w):
    return pl.pallas_call(
        _k1_body,
        grid=(3, 4),
        in_specs=[
            pl.BlockSpec((512, D), lambda j, i: (i, 0)),
            pl.BlockSpec((1, D), lambda j, i: (0, 0)),
            pl.BlockSpec((1, D), lambda j, i: (0, 0)),
            pl.BlockSpec((D, 1024), lambda j, i: (0, j)),
        ],
        out_specs=pl.BlockSpec((512, 1024), lambda j, i: (i, j)),
        out_shape=jax.ShapeDtypeStruct((N, 3 * D), jnp.float32),
    )(x, ln1_s.reshape(1, D), ln1_b.reshape(1, D), qkv_w)


def _k2_body(q_ref, k_ref, v_ref, o_ref):
    scale = HD ** -0.5
    for h in range(2):
        sl = slice(h * HD, (h + 1) * HD)
        q = q_ref[:, sl]
        k = k_ref[:, sl]
        v = v_ref[:, sl]
        s = lax.dot_general(q, k, (((1,), (1,)), ((), ())),
                            preferred_element_type=jnp.float32) * scale
        e = jnp.exp(s)
        r = 1.0 / jnp.sum(e, axis=-1, keepdims=True)
        o_ref[:, sl] = jnp.dot(e, v, preferred_element_type=jnp.float32) * r


def _k2(qkv):
    return pl.pallas_call(
        _k2_body,
        grid=(8, 4),
        in_specs=[
            pl.BlockSpec((512, 128), lambda p, t: (t, p)),
            pl.BlockSpec((N, 128), lambda p, t: (0, 8 + p)),
            pl.BlockSpec((N, 128), lambda p, t: (0, 16 + p)),
        ],
        out_specs=pl.BlockSpec((512, 128), lambda p, t: (t, p)),
        out_shape=jax.ShapeDtypeStruct((N, D), jnp.float32),
    )(qkv, qkv, qkv)


def _k3_body(o_ref, x_ref, pw_ref, pb_ref, s_ref, b_ref, gw_ref,
             x1_ref, t_ref, e0_ref, e1_ref, p0_ref, p1_ref, imp_ref, load_ref,
             r0_ref, r1_ref, cnt_ref):
    i = pl.program_id(0)
    x1 = x_ref[...] + jnp.dot(o_ref[...], pw_ref[...],
                              preferred_element_type=jnp.float32) + pb_ref[...]
    x1_ref[...] = x1
    mean = jnp.mean(x1, axis=-1, keepdims=True)
    var = jnp.mean((x1 - mean) ** 2, axis=-1, keepdims=True)
    t = (x1 - mean) * jax.lax.rsqrt(var + 1e-5) * s_ref[...] + b_ref[...]
    t_ref[...] = t
    logits = jnp.dot(t, gw_ref[...], preferred_element_type=jnp.float32)
    lane = lax.broadcasted_iota(jnp.int32, logits.shape, 1)
    v = jnp.where(lane < E, logits, NEG)
    m1 = jnp.max(v, axis=-1, keepdims=True)
    e0 = jnp.min(jnp.where(v == m1, lane, 128), axis=-1, keepdims=True)
    v2 = jnp.where(lane == e0, NEG, v)
    m2 = jnp.max(v2, axis=-1, keepdims=True)
    e1 = jnp.min(jnp.where(v2 == m2, lane, 128), axis=-1, keepdims=True)
    bexp = jnp.exp(m2 - m1)
    p0 = 1.0 / (1.0 + bexp)
    p1 = bexp / (1.0 + bexp)
    e0_ref[...] = e0.reshape(1, 256, 1)
    e1_ref[...] = e1.reshape(1, 256, 1)
    p0_ref[...] = p0.reshape(1, 256, 1)
    p1_ref[...] = p1.reshape(1, 256, 1)
    oh0 = jnp.where(lane == e0, 1.0, 0.0)
    oh1 = jnp.where(lane == e1, 1.0, 0.0)
    imp = jnp.sum(oh0 * p0 + oh1 * p1, axis=0, keepdims=True)
    ld = jnp.sum(oh0 * jnp.where(p0 > 0, 1.0, 0.0)
                 + oh1 * jnp.where(p1 > 0, 1.0, 0.0), axis=0, keepdims=True)

    @pl.when(i == 0)
    def _():
        imp_ref[...] = jnp.zeros_like(imp_ref)
        load_ref[...] = jnp.zeros_like(load_ref)
        cnt_ref[...] = jnp.zeros_like(cnt_ref)

    imp_ref[...] += imp
    load_ref[...] += ld
    carry = cnt_ref[...].astype(jnp.float32)
    oh = jnp.concatenate([oh0, oh1], axis=0)
    r_i = lax.broadcasted_iota(jnp.int32, (512, 512), 0)
    c_i = lax.broadcasted_iota(jnp.int32, (512, 512), 1)
    tri = jnp.where(r_i > c_i, 1.0, 0.0)
    cum_excl = jnp.dot(tri, oh, preferred_element_type=jnp.float32)
    rank = jnp.sum((cum_excl + carry) * oh, axis=-1, keepdims=True)
    r0_ref[...] = rank[:256].astype(jnp.int32).reshape(1, 256, 1)
    r1_ref[...] = rank[256:].astype(jnp.int32).reshape(1, 256, 1)
    cnt_ref[...] += jnp.sum(oh, axis=0, keepdims=True).astype(jnp.int32)


def _k3(o, x, proj_w, proj_b, ln2_s, ln2_b, gate_w):
    gw = jnp.pad(gate_w, ((0, 0), (0, 128 - E)))
    return pl.pallas_call(
        _k3_body,
        grid=(8,),
        in_specs=[
            pl.BlockSpec((256, D), lambda i: (i, 0)),
            pl.BlockSpec((256, D), lambda i: (i, 0)),
            pl.BlockSpec((D, D), lambda i: (0, 0)),
            pl.BlockSpec((1, D), lambda i: (0, 0)),
            pl.BlockSpec((1, D), lambda i: (0, 0)),
            pl.BlockSpec((1, D), lambda i: (0, 0)),
            pl.BlockSpec((D, 128), lambda i: (0, 0)),
        ],
        out_specs=[
            pl.BlockSpec((256, D), lambda i: (i, 0)),
            pl.BlockSpec((256, D), lambda i: (i, 0)),
            pl.BlockSpec((1, 256, 1), lambda i: (i, 0, 0)),
            pl.BlockSpec((1, 256, 1), lambda i: (i, 0, 0)),
            pl.BlockSpec((1, 256, 1), lambda i: (i, 0, 0)),
            pl.BlockSpec((1, 256, 1), lambda i: (i, 0, 0)),
            pl.BlockSpec((1, 128), lambda i: (0, 0)),
            pl.BlockSpec((1, 128), lambda i: (0, 0)),
            pl.BlockSpec((1, 256, 1), lambda i: (i, 0, 0)),
            pl.BlockSpec((1, 256, 1), lambda i: (i, 0, 0)),
            pl.BlockSpec((1, 128), lambda i: (0, 0)),
        ],
        out_shape=[
            jax.ShapeDtypeStruct((N, D), jnp.float32),
            jax.ShapeDtypeStruct((N, D), jnp.float32),
            jax.ShapeDtypeStruct((8, 256, 1), jnp.int32),
            jax.ShapeDtypeStruct((8, 256, 1), jnp.int32),
            jax.ShapeDtypeStruct((8, 256, 1), jnp.float32),
            jax.ShapeDtypeStruct((8, 256, 1), jnp.float32),
            jax.ShapeDtypeStruct((1, 128), jnp.float32),
            jax.ShapeDtypeStruct((1, 128), jnp.float32),
            jax.ShapeDtypeStruct((8, 256, 1), jnp.int32),
            jax.ShapeDtypeStruct((8, 256, 1), jnp.int32),
            jax.ShapeDtypeStruct((1, 128), jnp.int32),
        ],
    )(o, x, proj_w, proj_b.reshape(1, D), ln2_s.reshape(1, D),
      ln2_b.reshape(1, D), gw)


def _k3b_body(cnt_ref, e0_ref, e1_ref, r0_ref, r1_ref,
              d0_ref, d1_ref, texp_ref, meta_ref):
    i = pl.program_id(0)
    counts = cnt_ref[...]
    padf = (((counts + (TM - 1)) >> 8) << 8).astype(jnp.float32)
    r_i = lax.broadcasted_iota(jnp.int32, (128, 128), 0)
    c_i = lax.broadcasted_iota(jnp.int32, (128, 128), 1)
    tril = jnp.where(r_i < c_i, 1.0, 0.0)
    pstart = jnp.dot(padf, tril,
                     preferred_element_type=jnp.float32)
    e0 = e0_ref[...].reshape(256, 1)
    e1 = e1_ref[...].reshape(256, 1)
    lane = lax.broadcasted_iota(jnp.int32, (256, 128), 1)
    s0 = jnp.sum(jnp.where(lane == e0, pstart, 0.0), axis=-1, keepdims=True)
    s1 = jnp.sum(jnp.where(lane == e1, pstart, 0.0), axis=-1, keepdims=True)
    d0_ref[...] = (r0_ref[...].reshape(256, 1)
                   + s0.astype(jnp.int32)).reshape(1, 256, 1)
    d1_ref[...] = (r1_ref[...].reshape(256, 1)
                   + s1.astype(jnp.int32)).reshape(1, 256, 1)

    @pl.when(i == 0)
    def _():
        lanev = lax.broadcasted_iota(jnp.int32, (1, 128), 1)
        nv = jnp.sum(padf) * (1.0 / TM)
        last_e = jnp.max(jnp.where(counts > 0, lanev, 0))
        tile_row = (r_i * TM).astype(jnp.float32)
        pstart_b = jnp.broadcast_to(pstart, (128, 128))
        pend_b = pstart_b + jnp.broadcast_to(padf, (128, 128))
        inm = jnp.where((tile_row >= pstart_b) & (tile_row < pend_b), 1.0, 0.0)
        texp = jnp.sum(inm * c_i.astype(jnp.float32), axis=-1, keepdims=True)
        tcol = lax.broadcasted_iota(jnp.int32, (128, 1), 0).astype(jnp.float32)
        texp = jnp.where(tcol >= nv, last_e.astype(jnp.float32), texp)
        texp_ref[...] = texp.astype(jnp.int32)
        meta_ref[...] = jnp.where(lanev == 0, nv.astype(jnp.int32), 0)


def _k3b(cnt, e0c, e1c, r0c, r1c):
    return pl.pallas_call(
        _k3b_body,
        grid=(8,),
        in_specs=[
            pl.BlockSpec((1, 128), lambda i: (0, 0)),
            pl.BlockSpec((1, 256, 1), lambda i: (i, 0, 0)),
            pl.BlockSpec((1, 256, 1), lambda i: (i, 0, 0)),
            pl.BlockSpec((1, 256, 1), lambda i: (i, 0, 0)),
            pl.BlockSpec((1, 256, 1), lambda i: (i, 0, 0)),
        ],
        out_specs=[
            pl.BlockSpec((1, 256, 1), lambda i: (i, 0, 0)),
            pl.BlockSpec((1, 256, 1), lambda i: (i, 0, 0)),
            pl.BlockSpec((128, 1), lambda i: (0, 0)),
            pl.BlockSpec((1, 128), lambda i: (0, 0)),
        ],
        out_shape=[
            jax.ShapeDtypeStruct((8, 256, 1), jnp.int32),
            jax.ShapeDtypeStruct((8, 256, 1), jnp.int32),
            jax.ShapeDtypeStruct((128, 1), jnp.int32),
            jax.ShapeDtypeStruct((1, 128), jnp.int32),
        ],
    )(cnt, e0c, e1c, r0c, r1c)


def _mesh():
    return plsc.VectorSubcoreMesh(core_axis_name="c", subcore_axis_name="s")


def _k4_body(d0_hbm, d1_hbm, t_hbm, xg_hbm, idxv, rowsv, sem):
    wid = lax.axis_index("s") * 2 + lax.axis_index("c")
    base = wid * 64
    pltpu.sync_copy(t_hbm.at[pl.ds(base, 64)], rowsv)
    pltpu.sync_copy(d0_hbm.at[pl.ds(base, 64)], idxv)
    pltpu.async_copy(rowsv, xg_hbm.at[idxv], sem).wait()
    pltpu.sync_copy(d1_hbm.at[pl.ds(base, 64)], idxv)
    pltpu.async_copy(rowsv, xg_hbm.at[idxv], sem).wait()


def _k4(d0, d1, t):
    return pl.kernel(
        _k4_body,
        out_type=jax.ShapeDtypeStruct((P, D), jnp.float32),
        mesh=_mesh(),
        scratch_types=[
            pltpu.VMEM((64,), jnp.int32),
            pltpu.VMEM((64, D), jnp.float32),
            pltpu.SemaphoreType.DMA,
        ],
    )(d0, d1, t)


def _k5_body(texp_ref, meta_ref, xg_ref, w1_ref, b1_ref, w2_ref, b2_ref, yg_ref):
    i = pl.program_id(0)

    @pl.when(i < meta_ref[0])
    def _():
        xb = xg_ref[...].astype(jnp.bfloat16)
        h = jnp.dot(xb, w1_ref[0].astype(jnp.bfloat16),
                    preferred_element_type=jnp.float32) + b1_ref[0]
        hb = jax.nn.gelu(h).astype(jnp.bfloat16)
        yg_ref[...] = jnp.dot(hb, w2_ref[0].astype(jnp.bfloat16),
                              preferred_element_type=jnp.float32) + b2_ref[0]


def _k5(texp, meta, xg, w1, b1, w2, b2):
    def xg_idx(i, texp_ref, meta_ref):
        return (jnp.minimum(i, meta_ref[0] - 1), 0)

    def w_idx(i, texp_ref, meta_ref):
        return (texp_ref[jnp.minimum(i, meta_ref[0] - 1)], 0, 0)

    return pl.pallas_call(
        _k5_body,
        grid_spec=pltpu.PrefetchScalarGridSpec(
            num_scalar_prefetch=2,
            grid=(NT,),
            in_specs=[
                pl.BlockSpec((TM, D), xg_idx),
                pl.BlockSpec((1, D, HID), w_idx),
                pl.BlockSpec((1, 1, HID), w_idx),
                pl.BlockSpec((1, HID, D), w_idx),
                pl.BlockSpec((1, 1, D), w_idx),
            ],
            out_specs=pl.BlockSpec((TM, D), xg_idx),
        ),
        out_shape=jax.ShapeDtypeStruct((P, D), jnp.float32),
    )(texp, meta, xg, w1, b1.reshape(E, 1, HID), w2, b2.reshape(E, 1, D))


def _k6_body(d0_hbm, d1_hbm, yg_hbm, g0_hbm, g1_hbm, idxv, rowsv, sem):
    wid = lax.axis_index("s") * 2 + lax.axis_index("c")
    base = wid * 64
    pltpu.sync_copy(d0_hbm.at[pl.ds(base, 64)], idxv)
    pltpu.async_copy(yg_hbm.at[idxv], rowsv, sem).wait()
    pltpu.sync_copy(rowsv, g0_hbm.at[pl.ds(base, 64)])
    pltpu.sync_copy(d1_hbm.at[pl.ds(base, 64)], idxv)
    pltpu.async_copy(yg_hbm.at[idxv], rowsv, sem).wait()
    pltpu.sync_copy(rowsv, g1_hbm.at[pl.ds(base, 64)])


def _k6(d0, d1, yg):
    return pl.kernel(
        _k6_body,
        out_type=[
            jax.ShapeDtypeStruct((N, D), jnp.float32),
            jax.ShapeDtypeStruct((N, D), jnp.float32),
        ],
        mesh=_mesh(),
        scratch_types=[
            pltpu.VMEM((64,), jnp.int32),
            pltpu.VMEM((64, D), jnp.float32),
            pltpu.SemaphoreType.DMA,
        ],
    )(d0, d1, yg)


def _k7_body(x1_ref, g0_ref, g1_ref, p0_ref, p1_ref, out_ref):
    p0 = p0_ref[...].reshape(256, 1)
    p1 = p1_ref[...].reshape(256, 1)
    out_ref[...] = x1_ref[...] + p0 * g0_ref[...] + p1 * g1_ref[...]


def _k7(x1, g0, g1, p0c, p1c):
    return pl.pallas_call(
        _k7_body,
        grid=(8,),
        in_specs=[
            pl.BlockSpec((256, D), lambda i: (i, 0)),
            pl.BlockSpec((256, D), lambda i: (i, 0)),
            pl.BlockSpec((256, D), lambda i: (i, 0)),
            pl.BlockSpec((1, 256, 1), lambda i: (i, 0, 0)),
            pl.BlockSpec((1, 256, 1), lambda i: (i, 0, 0)),
        ],
        out_specs=pl.BlockSpec((256, D), lambda i: (i, 0)),
        out_shape=jax.ShapeDtypeStruct((N, D), jnp.float32),
    )(x1, g0, g1, p0c, p1c)


def kernel(x, ln1_scale, ln1_bias, qkv_w, proj_w, proj_b,
           ln2_scale, ln2_bias, gate_w, w1, b1, w2, b2):
    x2 = x.reshape(N, D)
    qkv = _k1(x2, ln1_scale, ln1_bias, qkv_w)
    o = _k2(qkv)
    x1, t, e0c, e1c, p0c, p1c, imp, load, r0c, r1c, cnt = _k3(
        o, x2, proj_w, proj_b, ln2_scale, ln2_bias, gate_w)
    d0c, d1c, texpc, metac = _k3b(cnt, e0c, e1c, r0c, r1c)
    d0 = d0c.reshape(N)
    d1 = d1c.reshape(N)
    texp = texpc.reshape(128)
    meta = metac.reshape(128)
    xg = _k4(d0, d1, t)
    yg = _k5(texp, meta, xg, w1, b1, w2, b2)
    g0, g1 = _k6(d0, d1, yg)
    xo = _k7(x1, g0, g1, p0c, p1c)
    return xo.reshape(1, N, D), imp[0, :E], load[0, :E]


# speedup vs baseline: 1.8757x; 1.0033x over previous
import jax
import jax.numpy as jnp
from jax import lax
from jax.experimental import pallas as pl
from jax.experimental.pallas import tpu as pltpu
from jax.experimental.pallas import tpu_sc as plsc

N = 2048
D = 1024
NH = 16
HD = 64
E = 8
TOPK = 2
HID = 1024
TM = 256
P = 6144
NT = P // TM
NEG = -1e30


def _k1_body(x_ref, s_ref, b_ref, w_ref, out_ref):
    x = x_ref[...]
    mean = jnp.mean(x, axis=-1, keepdims=True)
    var = jnp.mean((x - mean) ** 2, axis=-1, keepdims=True)
    h = (x - mean) * jax.lax.rsqrt(var + 1e-5) * s_ref[...] + b_ref[...]
    out_ref[...] = jnp.dot(h, w_ref[...], preferred_element_type=jnp.float32)


def _k1(x, ln1_s, ln1_b, qkv_w):
    return pl.pallas_call(
        _k1_body,
        grid=(3, 4),
        in_specs=[
            pl.BlockSpec((512, D), lambda j, i: (i, 0)),
            pl.BlockSpec((1, D), lambda j, i: (0, 0)),
            pl.BlockSpec((1, D), lambda j, i: (0, 0)),
            pl.BlockSpec((D, 1024), lambda j, i: (0, j)),
        ],
        out_specs=pl.BlockSpec((512, 1024), lambda j, i: (i, j)),
        out_shape=jax.ShapeDtypeStruct((N, 3 * D), jnp.float32),
    )(x, ln1_s.reshape(1, D), ln1_b.reshape(1, D), qkv_w)


def _k2_body(q_ref, k_ref, v_ref, o_ref):
    scale = HD ** -0.5
    for h in range(2):
        sl = slice(h * HD, (h + 1) * HD)
        q = q_ref[:, sl]
        k = k_ref[:, sl]
        v = v_ref[:, sl]
        s = lax.dot_general(q, k, (((1,), (1,)), ((), ())),
                            preferred_element_type=jnp.float32) * scale
        e = jnp.exp(s)
        r = 1.0 / jnp.sum(e, axis=-1, keepdims=True)
        o_ref[:, sl] = jnp.dot(e, v, preferred_element_type=jnp.float32) * r


def _k2(qkv):
    return pl.pallas_call(
        _k2_body,
        grid=(8, 4),
        in_specs=[
            pl.BlockSpec((512, 128), lambda p, t: (t, p)),
            pl.BlockSpec((N, 128), lambda p, t: (0, 8 + p)),
            pl.BlockSpec((N, 128), lambda p, t: (0, 16 + p)),
        ],
        out_specs=pl.BlockSpec((512, 128), lambda p, t: (t, p)),
        out_shape=jax.ShapeDtypeStruct((N, D), jnp.float32),
    )(qkv, qkv, qkv)


def _k3_body(o_ref, x_ref, pw_ref, pb_ref, s_ref, b_ref, gw_ref,
             x1_ref, t_ref, e0_ref, e1_ref, p0_ref, p1_ref, imp_ref, load_ref,
             r0_ref, r1_ref, cnt_ref):
    i = pl.program_id(0)
    x1 = x_ref[...] + jnp.dot(o_ref[...], pw_ref[...],
                              preferred_element_type=jnp.float32) + pb_ref[...]
    x1_ref[...] = x1
    mean = jnp.mean(x1, axis=-1, keepdims=True)
    var = jnp.mean((x1 - mean) ** 2, axis=-1, keepdims=True)
    t = (x1 - mean) * jax.lax.rsqrt(var + 1e-5) * s_ref[...] + b_ref[...]
    t_ref[...] = t
    logits = jnp.dot(t, gw_ref[...], preferred_element_type=jnp.float32)
    lane = lax.broadcasted_iota(jnp.int32, logits.shape, 1)
    v = jnp.where(lane < E, logits, NEG)
    m1 = jnp.max(v, axis=-1, keepdims=True)
    e0 = jnp.min(jnp.where(v == m1, lane, 128), axis=-1, keepdims=True)
    v2 = jnp.where(lane == e0, NEG, v)
    m2 = jnp.max(v2, axis=-1, keepdims=True)
    e1 = jnp.min(jnp.where(v2 == m2, lane, 128), axis=-1, keepdims=True)
    bexp = jnp.exp(m2 - m1)
    p0 = 1.0 / (1.0 + bexp)
    p1 = bexp / (1.0 + bexp)
    e0_ref[...] = e0.reshape(1, 256, 1)
    e1_ref[...] = e1.reshape(1, 256, 1)
    p0_ref[...] = p0.reshape(1, 256, 1)
    p1_ref[...] = p1.reshape(1, 256, 1)
    oh0 = jnp.where(lane == e0, 1.0, 0.0)
    oh1 = jnp.where(lane == e1, 1.0, 0.0)
    imp = jnp.sum(oh0 * p0 + oh1 * p1, axis=0, keepdims=True)
    ld = jnp.sum(oh0 * jnp.where(p0 > 0, 1.0, 0.0)
                 + oh1 * jnp.where(p1 > 0, 1.0, 0.0), axis=0, keepdims=True)

    @pl.when(i == 0)
    def _():
        imp_ref[...] = jnp.zeros_like(imp_ref)
        load_ref[...] = jnp.zeros_like(load_ref)
        cnt_ref[...] = jnp.zeros_like(cnt_ref)

    imp_ref[...] += imp
    load_ref[...] += ld
    carry = cnt_ref[...].astype(jnp.float32)
    oh = jnp.concatenate([oh0, oh1], axis=0)
    r_i = lax.broadcasted_iota(jnp.int32, (512, 512), 0)
    c_i = lax.broadcasted_iota(jnp.int32, (512, 512), 1)
    tri = jnp.where(r_i > c_i, 1.0, 0.0)
    cum_excl = jnp.dot(tri, oh, preferred_element_type=jnp.float32)
    rank = jnp.sum((cum_excl + carry) * oh, axis=-1, keepdims=True)
    r0_ref[...] = rank[:256].astype(jnp.int32).reshape(1, 256, 1)
    r1_ref[...] = rank[256:].astype(jnp.int32).reshape(1, 256, 1)
    cnt_ref[...] += jnp.sum(oh, axis=0, keepdims=True).astype(jnp.int32)


def _k3(o, x, proj_w, proj_b, ln2_s, ln2_b, gate_w):
    gw = jnp.pad(gate_w, ((0, 0), (0, 128 - E)))
    return pl.pallas_call(
        _k3_body,
        grid=(8,),
        in_specs=[
            pl.BlockSpec((256, D), lambda i: (i, 0)),
            pl.BlockSpec((256, D), lambda i: (i, 0)),
            pl.BlockSpec((D, D), lambda i: (0, 0)),
            pl.BlockSpec((1, D), lambda i: (0, 0)),
            pl.BlockSpec((1, D), lambda i: (0, 0)),
            pl.BlockSpec((1, D), lambda i: (0, 0)),
            pl.BlockSpec((D, 128), lambda i: (0, 0)),
        ],
        out_specs=[
            pl.BlockSpec((256, D), lambda i: (i, 0)),
            pl.BlockSpec((256, D), lambda i: (i, 0)),
            pl.BlockSpec((1, 256, 1), lambda i: (i, 0, 0)),
            pl.BlockSpec((1, 256, 1), lambda i: (i, 0, 0)),
            pl.BlockSpec((1, 256, 1), lambda i: (i, 0, 0)),
            pl.BlockSpec((1, 256, 1), lambda i: (i, 0, 0)),
            pl.BlockSpec((1, 128), lambda i: (0, 0)),
            pl.BlockSpec((1, 128), lambda i: (0, 0)),
            pl.BlockSpec((1, 256, 1), lambda i: (i, 0, 0)),
            pl.BlockSpec((1, 256, 1), lambda i: (i, 0, 0)),
            pl.BlockSpec((1, 128), lambda i: (0, 0)),
        ],
        out_shape=[
            jax.ShapeDtypeStruct((N, D), jnp.float32),
            jax.ShapeDtypeStruct((N, D), jnp.float32),
            jax.ShapeDtypeStruct((8, 256, 1), jnp.int32),
            jax.ShapeDtypeStruct((8, 256, 1), jnp.int32),
            jax.ShapeDtypeStruct((8, 256, 1), jnp.float32),
            jax.ShapeDtypeStruct((8, 256, 1), jnp.float32),
            jax.ShapeDtypeStruct((1, 128), jnp.float32),
            jax.ShapeDtypeStruct((1, 128), jnp.float32),
            jax.ShapeDtypeStruct((8, 256, 1), jnp.int32),
            jax.ShapeDtypeStruct((8, 256, 1), jnp.int32),
            jax.ShapeDtypeStruct((1, 128), jnp.int32),
        ],
    )(o, x, proj_w, proj_b.reshape(1, D), ln2_s.reshape(1, D),
      ln2_b.reshape(1, D), gw)


def _k3b_body(cnt_ref, e0_ref, e1_ref, r0_ref, r1_ref,
              d0_ref, d1_ref, texp_ref, meta_ref):
    i = pl.program_id(0)
    counts = cnt_ref[...]
    padf = (((counts + (TM - 1)) >> 8) << 8).astype(jnp.float32)
    r_i = lax.broadcasted_iota(jnp.int32, (128, 128), 0)
    c_i = lax.broadcasted_iota(jnp.int32, (128, 128), 1)
    tril = jnp.where(r_i < c_i, 1.0, 0.0)
    pstart = jnp.dot(padf, tril,
                     preferred_element_type=jnp.float32)
    e0 = e0_ref[...].reshape(256, 1)
    e1 = e1_ref[...].reshape(256, 1)
    lane = lax.broadcasted_iota(jnp.int32, (256, 128), 1)
    s0 = jnp.sum(jnp.where(lane == e0, pstart, 0.0), axis=-1, keepdims=True)
    s1 = jnp.sum(jnp.where(lane == e1, pstart, 0.0), axis=-1, keepdims=True)
    d0_ref[...] = (r0_ref[...].reshape(256, 1)
                   + s0.astype(jnp.int32)).reshape(1, 256, 1)
    d1_ref[...] = (r1_ref[...].reshape(256, 1)
                   + s1.astype(jnp.int32)).reshape(1, 256, 1)

    @pl.when(i == 0)
    def _():
        lanev = lax.broadcasted_iota(jnp.int32, (1, 128), 1)
        nv = jnp.sum(padf) * (1.0 / TM)
        last_e = jnp.max(jnp.where(counts > 0, lanev, 0))
        tile_row = (r_i * TM).astype(jnp.float32)
        pstart_b = jnp.broadcast_to(pstart, (128, 128))
        pend_b = pstart_b + jnp.broadcast_to(padf, (128, 128))
        inm = jnp.where((tile_row >= pstart_b) & (tile_row < pend_b), 1.0, 0.0)
        texp = jnp.sum(inm * c_i.astype(jnp.float32), axis=-1, keepdims=True)
        tcol = lax.broadcasted_iota(jnp.int32, (128, 1), 0).astype(jnp.float32)
        texp = jnp.where(tcol >= nv, last_e.astype(jnp.float32), texp)
        texp_ref[...] = texp.astype(jnp.int32)
        meta_ref[...] = jnp.where(lanev == 0, nv.astype(jnp.int32), 0)


def _k3b(cnt, e0c, e1c, r0c, r1c):
    return pl.pallas_call(
        _k3b_body,
        grid=(8,),
        in_specs=[
            pl.BlockSpec((1, 128), lambda i: (0, 0)),
            pl.BlockSpec((1, 256, 1), lambda i: (i, 0, 0)),
            pl.BlockSpec((1, 256, 1), lambda i: (i, 0, 0)),
            pl.BlockSpec((1, 256, 1), lambda i: (i, 0, 0)),
            pl.BlockSpec((1, 256, 1), lambda i: (i, 0, 0)),
        ],
        out_specs=[
            pl.BlockSpec((1, 256, 1), lambda i: (i, 0, 0)),
            pl.BlockSpec((1, 256, 1), lambda i: (i, 0, 0)),
            pl.BlockSpec((128, 1), lambda i: (0, 0)),
            pl.BlockSpec((1, 128), lambda i: (0, 0)),
        ],
        out_shape=[
            jax.ShapeDtypeStruct((8, 256, 1), jnp.int32),
            jax.ShapeDtypeStruct((8, 256, 1), jnp.int32),
            jax.ShapeDtypeStruct((128, 1), jnp.int32),
            jax.ShapeDtypeStruct((1, 128), jnp.int32),
        ],
    )(cnt, e0c, e1c, r0c, r1c)


def _mesh():
    return plsc.VectorSubcoreMesh(core_axis_name="c", subcore_axis_name="s")


def _k4_body(d0_hbm, d1_hbm, t_hbm, xg_hbm, idx0v, idx1v, rowsv, sem):
    wid = lax.axis_index("s") * 2 + lax.axis_index("c")
    base = wid * 64
    cr = pltpu.async_copy(t_hbm.at[pl.ds(base, 64)], rowsv, sem)
    pltpu.sync_copy(d0_hbm.at[pl.ds(base, 64)], idx0v)
    pltpu.sync_copy(d1_hbm.at[pl.ds(base, 64)], idx1v)
    cr.wait()
    c0 = pltpu.async_copy(rowsv, xg_hbm.at[idx0v], sem)
    c1 = pltpu.async_copy(rowsv, xg_hbm.at[idx1v], sem)
    c0.wait()
    c1.wait()


def _k4(d0, d1, t):
    return pl.kernel(
        _k4_body,
        out_type=jax.ShapeDtypeStruct((P, D), jnp.float32),
        mesh=_mesh(),
        scratch_types=[
            pltpu.VMEM((64,), jnp.int32),
            pltpu.VMEM((64,), jnp.int32),
            pltpu.VMEM((64, D), jnp.float32),
            pltpu.SemaphoreType.DMA,
        ],
    )(d0, d1, t)


def _k5_body(texp_ref, meta_ref, xg_ref, w1_ref, b1_ref, w2_ref, b2_ref, yg_ref):
    i = pl.program_id(0)

    @pl.when(i < meta_ref[0])
    def _():
        xb = xg_ref[...].astype(jnp.bfloat16)
        h = jnp.dot(xb, w1_ref[0].astype(jnp.bfloat16),
                    preferred_element_type=jnp.float32) + b1_ref[0]
        hb = jax.nn.gelu(h).astype(jnp.bfloat16)
        yg_ref[...] = jnp.dot(hb, w2_ref[0].astype(jnp.bfloat16),
                              preferred_element_type=jnp.float32) + b2_ref[0]


def _k5(texp, meta, xg, w1, b1, w2, b2):
    def xg_idx(i, texp_ref, meta_ref):
        return (jnp.minimum(i, meta_ref[0] - 1), 0)

    def w_idx(i, texp_ref, meta_ref):
        return (texp_ref[jnp.minimum(i, meta_ref[0] - 1)], 0, 0)

    return pl.pallas_call(
        _k5_body,
        grid_spec=pltpu.PrefetchScalarGridSpec(
            num_scalar_prefetch=2,
            grid=(NT,),
            in_specs=[
                pl.BlockSpec((TM, D), xg_idx),
                pl.BlockSpec((1, D, HID), w_idx),
                pl.BlockSpec((1, 1, HID), w_idx),
                pl.BlockSpec((1, HID, D), w_idx),
                pl.BlockSpec((1, 1, D), w_idx),
            ],
            out_specs=pl.BlockSpec((TM, D), xg_idx),
        ),
        out_shape=jax.ShapeDtypeStruct((P, D), jnp.float32),
    )(texp, meta, xg, w1, b1.reshape(E, 1, HID), w2, b2.reshape(E, 1, D))


def _k6_body(d0_hbm, d1_hbm, yg_hbm, g0_hbm, g1_hbm,
             idx0v, idx1v, rows0v, rows1v, sem):
    wid = lax.axis_index("s") * 2 + lax.axis_index("c")
    base = wid * 64
    pltpu.sync_copy(d0_hbm.at[pl.ds(base, 64)], idx0v)
    pltpu.sync_copy(d1_hbm.at[pl.ds(base, 64)], idx1v)
    for ch in range(2):
        cb = base + ch * 32
        c0 = pltpu.async_copy(yg_hbm.at[idx0v.at[pl.ds(ch * 32, 32)]],
                              rows0v, sem)
        c1 = pltpu.async_copy(yg_hbm.at[idx1v.at[pl.ds(ch * 32, 32)]],
                              rows1v, sem)
        c0.wait()
        pltpu.sync_copy(rows0v, g0_hbm.at[pl.ds(cb, 32)])
        c1.wait()
        pltpu.sync_copy(rows1v, g1_hbm.at[pl.ds(cb, 32)])


def _k6(d0, d1, yg):
    return pl.kernel(
        _k6_body,
        out_type=[
            jax.ShapeDtypeStruct((N, D), jnp.float32),
            jax.ShapeDtypeStruct((N, D), jnp.float32),
        ],
        mesh=_mesh(),
        scratch_types=[
            pltpu.VMEM((64,), jnp.int32),
            pltpu.VMEM((64,), jnp.int32),
            pltpu.VMEM((32, D), jnp.float32),
            pltpu.VMEM((32, D), jnp.float32),
            pltpu.SemaphoreType.DMA,
        ],
    )(d0, d1, yg)


def _k7_body(x1_ref, g0_ref, g1_ref, p0_ref, p1_ref, out_ref):
    p0 = p0_ref[...].reshape(256, 1)
    p1 = p1_ref[...].reshape(256, 1)
    out_ref[...] = x1_ref[...] + p0 * g0_ref[...] + p1 * g1_ref[...]


def _k7(x1, g0, g1, p0c, p1c):
    return pl.pallas_call(
        _k7_body,
        grid=(8,),
        in_specs=[
            pl.BlockSpec((256, D), lambda i: (i, 0)),
            pl.BlockSpec((256, D), lambda i: (i, 0)),
            pl.BlockSpec((256, D), lambda i: (i, 0)),
            pl.BlockSpec((1, 256, 1), lambda i: (i, 0, 0)),
            pl.BlockSpec((1, 256, 1), lambda i: (i, 0, 0)),
        ],
        out_specs=pl.BlockSpec((256, D), lambda i: (i, 0)),
        out_shape=jax.ShapeDtypeStruct((N, D), jnp.float32),
    )(x1, g0, g1, p0c, p1c)


def kernel(x, ln1_scale, ln1_bias, qkv_w, proj_w, proj_b,
           ln2_scale, ln2_bias, gate_w, w1, b1, w2, b2):
    x2 = x.reshape(N, D)
    qkv = _k1(x2, ln1_scale, ln1_bias, qkv_w)
    o = _k2(qkv)
    x1, t, e0c, e1c, p0c, p1c, imp, load, r0c, r1c, cnt = _k3(
        o, x2, proj_w, proj_b, ln2_scale, ln2_bias, gate_w)
    d0c, d1c, texpc, metac = _k3b(cnt, e0c, e1c, r0c, r1c)
    d0 = d0c.reshape(N)
    d1 = d1c.reshape(N)
    texp = texpc.reshape(128)
    meta = metac.reshape(128)
    xg = _k4(d0, d1, t)
    yg = _k5(texp, meta, xg, w1, b1, w2, b2)
    g0, g1 = _k6(d0, d1, yg)
    xo = _k7(x1, g0, g1, p0c, p1c)
    return xo.reshape(1, N, D), imp[0, :E], load[0, :E]
